# Initial kernel scaffold; baseline (speedup 1.0000x reference)
import jax
import jax.numpy as jnp
from jax.experimental import pallas as pl


def kernel(x, edge_index, batch, W_l0, b_l0, W_r0, W_l1, b_l1, W_r1):
    raise NotImplementedError("write your pallas kernel here")


# speedup vs baseline: 3.0991x; 3.0991x over previous
import jax
import jax.numpy as jnp
from jax import lax
from jax.experimental import pallas as pl
from jax.experimental.pallas import tpu as pltpu
from jax.experimental.pallas import tpu_sc as plsc

_N = 10000
_E = 320000
_D = 128
_G = 64
_NPAD = 10240
_NSUB = 16
_NCORE = 2
_NW = _NCORE * _NSUB
_CH = 128
_EPW = 10240
_NCH = _EPW // _CH
_RPT = _NPAD // _NSUB

_F32 = jnp.float32


def _sc_agg_body(feat_hbm, srcp_hbm, dstp_hbm, ones_hbm, zr_hbm, zc_hbm,
                 sum_out, cnt_out,
                 src_all, dst_all, rbuf, ones_v, gsem, acc, cntacc):
    c = lax.axis_index("c")
    s = lax.axis_index("s")
    wid = c * _NSUB + s
    pltpu.sync_copy(srcp_hbm.at[wid], src_all)
    pltpu.sync_copy(dstp_hbm.at[wid], dst_all)
    pltpu.sync_copy(ones_hbm, ones_v)
    base = s * _RPT
    pltpu.sync_copy(zr_hbm, acc.at[pl.ds(base, _RPT)])
    pltpu.sync_copy(zc_hbm, cntacc.at[pl.ds(base, _RPT)])
    plsc.subcore_barrier()

    def chunk(j, carry):
        pltpu.async_copy(feat_hbm.at[src_all.at[j]], rbuf, gsem).wait()
        pltpu.sync_copy(rbuf, acc.at[dst_all.at[j]], add=True)
        pltpu.sync_copy(ones_v, cntacc.at[dst_all.at[j]], add=True)
        return carry

    lax.fori_loop(0, _NCH, chunk, 0)
    plsc.subcore_barrier()
    pltpu.sync_copy(acc.at[pl.ds(base, _RPT)], sum_out.at[c, pl.ds(base, _RPT)])
    pltpu.sync_copy(cntacc.at[pl.ds(base, _RPT)], cnt_out.at[c, pl.ds(base, _RPT)])


_sc_agg = pl.kernel(
    _sc_agg_body,
    out_type=[jax.ShapeDtypeStruct((_NCORE, _NPAD, _D), _F32),
              jax.ShapeDtypeStruct((_NCORE, _NPAD), _F32)],
    mesh=plsc.VectorSubcoreMesh(core_axis_name="c", subcore_axis_name="s"),
    scratch_types=[
        pltpu.VMEM((_NCH, _CH), jnp.int32),
        pltpu.VMEM((_NCH, _CH), jnp.int32),
        pltpu.VMEM((_CH, _D), _F32),
        pltpu.VMEM((_CH,), _F32),
        pltpu.SemaphoreType.DMA,
        pltpu.VMEM_SHARED((_NPAD, _D), _F32),
        pltpu.VMEM_SHARED((_NPAD,), _F32),
    ],
)


_R = 1024

def _dot(a, b):
    return jnp.dot(a, b, preferred_element_type=_F32,
                   precision=lax.Precision.HIGHEST)


def _tc1_body(s0, s1, c0, c1, x, wl, bb, wr, o):
    cnt = jnp.maximum(c0[...] + c1[...], 1.0)
    mean = (s0[...] + s1[...]) / cnt
    h = _dot(mean, wl[...]) + bb[...] + _dot(x[...], wr[...])
    o[...] = jnp.maximum(h, 0.0)


def _tc1(s0, s1, c0, c1, xpad, wlT, b, wrT):
    bs_r = pl.BlockSpec((_R, _D), lambda i: (i, 0))
    bs_c = pl.BlockSpec((_R, 1), lambda i: (i, 0))
    bs_w = pl.BlockSpec((_D, _D), lambda i: (0, 0))
    bs_b = pl.BlockSpec((1, _D), lambda i: (0, 0))
    return pl.pallas_call(
        _tc1_body,
        grid=(_NPAD // _R,),
        in_specs=[bs_r, bs_r, bs_c, bs_c, bs_r, bs_w, bs_b, bs_w],
        out_specs=bs_r,
        out_shape=jax.ShapeDtypeStruct((_NPAD, _D), _F32),
    )(s0, s1, c0, c1, xpad, wlT, b, wrT)


def _firsts_body(b_ref, f_ref):
    g = lax.broadcasted_iota(jnp.int32, (1, _G), 1)
    lt = (b_ref[...] < g).astype(jnp.int32)
    f = jnp.sum(lt, axis=0, keepdims=True)
    f_ref[...] = jnp.minimum(f, _N - 1)


def _firsts(batchp):
    return pl.pallas_call(
        _firsts_body,
        grid=(1,),
        in_specs=[pl.BlockSpec((_NPAD, 1), lambda i: (0, 0))],
        out_specs=pl.BlockSpec((1, _G), lambda i: (0, 0)),
        out_shape=jax.ShapeDtypeStruct((1, _G), jnp.int32),
    )(batchp)


def _tcf_body(f, s0, s1, c0, c1, h, wl, bb, wr, o, accm, acch):
    i = pl.program_id(0)

    @pl.when(i == 0)
    def _init():
        accm[...] = jnp.zeros_like(accm)
        acch[...] = jnp.zeros_like(acch)

    rows = lax.broadcasted_iota(jnp.int32, (_R, 1), 0) + i * _R
    oh = (rows == f[...]).astype(_F32)
    cnt = jnp.maximum(c0[...] + c1[...], 1.0)
    mean = (s0[...] + s1[...]) / cnt

    def gather(a, b):
        return lax.dot_general(a, b, (((0,), (0,)), ((), ())),
                               preferred_element_type=_F32,
                               precision=lax.Precision.HIGHEST)

    accm[...] += gather(oh, mean)
    acch[...] += gather(oh, h[...])

    @pl.when(i == _NPAD // _R - 1)
    def _fin():
        o[...] = _dot(accm[...], wl[...]) + bb[...] + _dot(acch[...], wr[...])


def _tcf(f, s0, s1, c0, c1, h, wlT, b, wrT):
    bs_r = pl.BlockSpec((_R, _D), lambda i: (i, 0))
    bs_c = pl.BlockSpec((_R, 1), lambda i: (i, 0))
    bs_w = pl.BlockSpec((_D, _D), lambda i: (0, 0))
    bs_b = pl.BlockSpec((1, _D), lambda i: (0, 0))
    bs_f = pl.BlockSpec((1, _G), lambda i: (0, 0))
    return pl.pallas_call(
        _tcf_body,
        grid=(_NPAD // _R,),
        in_specs=[bs_f, bs_r, bs_r, bs_c, bs_c, bs_r, bs_w, bs_b, bs_w],
        out_specs=pl.BlockSpec((_G, _D), lambda i: (0, 0)),
        out_shape=jax.ShapeDtypeStruct((_G, _D), _F32),
        scratch_shapes=[pltpu.VMEM((_G, _D), _F32),
                        pltpu.VMEM((_G, _D), _F32)],
    )(f, s0, s1, c0, c1, h, wlT, b, wrT)


def kernel(x, edge_index, batch, W_l0, b_l0, W_r0, W_l1, b_l1, W_r1):
    src = edge_index[0]
    dst = edge_index[1]
    padlen = _NW * _EPW - _E
    srcp = jnp.concatenate([src, jnp.zeros((padlen,), jnp.int32)]
                           ).reshape(_NW, _NCH, _CH)
    dstp = jnp.concatenate([dst, jnp.full((padlen,), _N, jnp.int32)]
                           ).reshape(_NW, _NCH, _CH)
    ones = jnp.ones((_CH,), _F32)
    zr = jnp.zeros((_RPT, _D), _F32)
    zc = jnp.zeros((_RPT,), _F32)
    xpad = jnp.pad(x, ((0, _NPAD - _N), (0, 0)))
    batchp = jnp.pad(batch, (0, _NPAD - _N),
                     constant_values=_G - 1).reshape(_NPAD, 1)

    sum1, cnt1 = _sc_agg(xpad, srcp, dstp, ones, zr, zc)
    h = _tc1(sum1[0], sum1[1],
             cnt1[0].reshape(_NPAD, 1), cnt1[1].reshape(_NPAD, 1),
             xpad, W_l0.T, b_l0.reshape(1, _D), W_r0.T)
    sum2, cnt2 = _sc_agg(h, srcp, dstp, ones, zr, zc)
    f = _firsts(batchp)
    return _tcf(f, sum2[0], sum2[1],
                cnt2[0].reshape(_NPAD, 1), cnt2[1].reshape(_NPAD, 1),
                h, W_l1.T, b_l1.reshape(1, _D), W_r1.T)


# speedup vs baseline: 5.1760x; 1.6702x over previous
import jax
import jax.numpy as jnp
from jax import lax
from jax.experimental import pallas as pl
from jax.experimental.pallas import tpu as pltpu
from jax.experimental.pallas import tpu_sc as plsc

_N = 10000
_E = 320000
_D = 128
_G = 64
_NPAD = 10240
_NSUB = 16
_NCORE = 2
_NW = _NCORE * _NSUB
_CH = 128
_EPW = 10240
_NCH = _EPW // _CH
_RPT = _NPAD // _NSUB

_F32 = jnp.float32


def _sc_agg_body(feat_hbm, srcp_hbm, dstp_hbm, ones_hbm, zr_hbm, zc_hbm,
                 sum_out, cnt_out,
                 src_all, dst_all, rbuf, ones_v, gsem, acc, cntacc):
    c = lax.axis_index("c")
    s = lax.axis_index("s")
    wid = c * _NSUB + s
    pltpu.sync_copy(srcp_hbm.at[wid], src_all)
    pltpu.sync_copy(dstp_hbm.at[wid], dst_all)
    pltpu.sync_copy(ones_hbm, ones_v)
    base = s * _RPT
    pltpu.sync_copy(zr_hbm, acc.at[pl.ds(base, _RPT)])
    pltpu.sync_copy(zc_hbm, cntacc.at[pl.ds(base, _RPT)])
    plsc.subcore_barrier()

    def chunk(j, carry):
        pltpu.async_copy(feat_hbm.at[src_all.at[j]], rbuf, gsem).wait()
        pltpu.sync_copy(rbuf, acc.at[dst_all.at[j]], add=True)
        pltpu.sync_copy(ones_v, cntacc.at[dst_all.at[j]], add=True)
        return carry

    lax.fori_loop(0, _NCH, chunk, 0)
    plsc.subcore_barrier()
    pltpu.sync_copy(acc.at[pl.ds(base, _RPT)], sum_out.at[c, pl.ds(base, _RPT)])
    pltpu.sync_copy(cntacc.at[pl.ds(base, _RPT)], cnt_out.at[c, pl.ds(base, _RPT)])


_SL = 128


def _sc_agg2_body(feat_hbm, srci_hbm, dsti_hbm, slot_hbm, ones_hbm,
                  zr_hbm, zc_hbm,
                  sum_out, cnt_out,
                  srcf, dstf, slottab, fsrcf, fslotf, sidx, rbuf, ones_v,
                  gsem, acc, cntacc):
    c = lax.axis_index("c")
    s = lax.axis_index("s")
    wid = c * _NSUB + s
    pltpu.sync_copy(srci_hbm.at[wid], srcf)
    pltpu.sync_copy(dsti_hbm.at[wid], dstf)
    pltpu.sync_copy(slot_hbm, slottab)
    pltpu.sync_copy(ones_hbm, ones_v)

    @pl.when(s == 0)
    def _zero():
        pltpu.sync_copy(zr_hbm.at[pl.ds(0, _SL)], acc)
        pltpu.sync_copy(zc_hbm.at[pl.ds(0, _SL)], cntacc)

    def pre(i, carry):
        fsrcf[pl.ds(i * 16, 16)] = jnp.zeros((16,), jnp.int32)
        fslotf[pl.ds(i * 16, 16)] = jnp.full((16,), _G, jnp.int32)
        return carry

    lax.fori_loop(0, (_EPW + 16) // 16, pre, 0)
    plsc.subcore_barrier()

    def scan(i, fcnt):
        dv = dstf[pl.ds(i * 16, 16)]
        sl = plsc.load_gather(slottab, [dv])
        m = sl >= 0
        sv = srcf[pl.ds(i * 16, 16)]
        plsc.store_compressed(fsrcf.at[pl.ds(fcnt, 16)], sv, mask=m)
        plsc.store_compressed(fslotf.at[pl.ds(fcnt, 16)], sl, mask=m)
        return fcnt + jnp.sum(m.astype(jnp.int32))

    fcnt = lax.fori_loop(0, _EPW // 16, scan, 0)
    nch = (fcnt + _CH - 1) // _CH

    def chunk(j, carry):
        for k in range(8):
            sidx[pl.ds(k * 16, 16)] = fslotf[pl.ds(j * _CH + k * 16, 16)]
        pltpu.async_copy(feat_hbm.at[fsrcf.at[pl.ds(j * _CH, _CH)]],
                         rbuf, gsem).wait()
        pltpu.sync_copy(rbuf, acc.at[sidx], add=True)
        pltpu.sync_copy(ones_v, cntacc.at[sidx], add=True)
        return carry

    lax.fori_loop(0, nch, chunk, 0)
    plsc.subcore_barrier()

    @pl.when(s == 0)
    def _write():
        pltpu.sync_copy(acc, sum_out.at[c])
        pltpu.sync_copy(cntacc, cnt_out.at[c])


_sc_agg2 = pl.kernel(
    _sc_agg2_body,
    out_type=[jax.ShapeDtypeStruct((_NCORE, _SL, _D), _F32),
              jax.ShapeDtypeStruct((_NCORE, _SL), _F32)],
    mesh=plsc.VectorSubcoreMesh(core_axis_name="c", subcore_axis_name="s"),
    scratch_types=[
        pltpu.VMEM((_EPW,), jnp.int32),
        pltpu.VMEM((_EPW,), jnp.int32),
        pltpu.VMEM((_NPAD,), jnp.int32),
        pltpu.VMEM((_EPW + 16,), jnp.int32),
        pltpu.VMEM((_EPW + 16,), jnp.int32),
        pltpu.VMEM((_CH,), jnp.int32),
        pltpu.VMEM((_CH, _D), _F32),
        pltpu.VMEM((_CH,), _F32),
        pltpu.SemaphoreType.DMA,
        pltpu.VMEM_SHARED((_SL, _D), _F32),
        pltpu.VMEM_SHARED((_SL,), _F32),
    ],
    compiler_params=pltpu.CompilerParams(needs_layout_passes=False),
)


_sc_agg = pl.kernel(
    _sc_agg_body,
    out_type=[jax.ShapeDtypeStruct((_NCORE, _NPAD, _D), _F32),
              jax.ShapeDtypeStruct((_NCORE, _NPAD), _F32)],
    mesh=plsc.VectorSubcoreMesh(core_axis_name="c", subcore_axis_name="s"),
    scratch_types=[
        pltpu.VMEM((_NCH, _CH), jnp.int32),
        pltpu.VMEM((_NCH, _CH), jnp.int32),
        pltpu.VMEM((_CH, _D), _F32),
        pltpu.VMEM((_CH,), _F32),
        pltpu.SemaphoreType.DMA,
        pltpu.VMEM_SHARED((_NPAD, _D), _F32),
        pltpu.VMEM_SHARED((_NPAD,), _F32),
    ],
    compiler_params=pltpu.CompilerParams(needs_layout_passes=False),
)


_R = 1024

def _dot(a, b):
    return jnp.dot(a, b, preferred_element_type=_F32,
                   precision=lax.Precision.HIGHEST)


def _tc1_body(s0, s1, c0, c1, x, wl, bb, wr, o):
    cnt = jnp.maximum(c0[...] + c1[...], 1.0)
    mean = (s0[...] + s1[...]) / cnt
    h = _dot(mean, wl[...]) + bb[...] + _dot(x[...], wr[...])
    o[...] = jnp.maximum(h, 0.0)


def _tc1(s0, s1, c0, c1, xpad, wlT, b, wrT):
    bs_r = pl.BlockSpec((_R, _D), lambda i: (i, 0))
    bs_c = pl.BlockSpec((_R, 1), lambda i: (i, 0))
    bs_w = pl.BlockSpec((_D, _D), lambda i: (0, 0))
    bs_b = pl.BlockSpec((1, _D), lambda i: (0, 0))
    return pl.pallas_call(
        _tc1_body,
        grid=(_NPAD // _R,),
        in_specs=[bs_r, bs_r, bs_c, bs_c, bs_r, bs_w, bs_b, bs_w],
        out_specs=bs_r,
        out_shape=jax.ShapeDtypeStruct((_NPAD, _D), _F32),
    )(s0, s1, c0, c1, xpad, wlT, b, wrT)


def _firsts_body(b_ref, f_ref, slot_ref):
    g = lax.broadcasted_iota(jnp.int32, (1, _G), 1)
    lt = (b_ref[...] < g).astype(jnp.int32)
    f = jnp.minimum(jnp.sum(lt, axis=0, keepdims=True), _N - 1)
    f_ref[...] = f
    rows = lax.broadcasted_iota(jnp.int32, (_NPAD, 1), 0)
    cand = jnp.where(rows == f, g, _G)
    sl = jnp.min(cand, axis=1, keepdims=True)
    slot_ref[...] = jnp.where(sl == _G, -1, sl)


def _firsts(batchp):
    return pl.pallas_call(
        _firsts_body,
        grid=(1,),
        in_specs=[pl.BlockSpec((_NPAD, 1), lambda i: (0, 0))],
        out_specs=[pl.BlockSpec((1, _G), lambda i: (0, 0)),
                   pl.BlockSpec((_NPAD, 1), lambda i: (0, 0))],
        out_shape=[jax.ShapeDtypeStruct((1, _G), jnp.int32),
                   jax.ShapeDtypeStruct((_NPAD, 1), jnp.int32)],
    )(batchp)


def _tcf_body(f, s0, s1, c0, c1, h, wl, bb, wr, o, acch):
    i = pl.program_id(0)

    @pl.when(i == 0)
    def _init():
        acch[...] = jnp.zeros_like(acch)

    rows = lax.broadcasted_iota(jnp.int32, (_R, 1), 0) + i * _R
    oh = (rows == f[...]).astype(_F32)
    acch[...] += lax.dot_general(oh, h[...], (((0,), (0,)), ((), ())),
                                 preferred_element_type=_F32,
                                 precision=lax.Precision.HIGHEST)

    @pl.when(i == _NPAD // _R - 1)
    def _fin():
        fv = f[...]
        gp = lax.broadcasted_iota(jnp.int32, (_G, _G), 1)
        eqm = jnp.reshape(fv, (_G, 1)) == fv
        dmin = jnp.min(jnp.where(eqm, gp, _G), axis=1, keepdims=True)
        P = (gp == dmin).astype(_F32)
        cnt = jnp.maximum(c0[...] + c1[...], 1.0)
        mean = (s0[...] + s1[...]) / cnt
        mean_sel = _dot(P, mean)
        o[...] = _dot(mean_sel, wl[...]) + bb[...] + _dot(acch[...], wr[...])


def _tcf(f, s0, s1, c0, c1, h, wlT, b, wrT):
    bs_r = pl.BlockSpec((_R, _D), lambda i: (i, 0))
    bs_g = pl.BlockSpec((_G, _D), lambda i: (0, 0))
    bs_c = pl.BlockSpec((_G, 1), lambda i: (0, 0))
    bs_w = pl.BlockSpec((_D, _D), lambda i: (0, 0))
    bs_b = pl.BlockSpec((1, _D), lambda i: (0, 0))
    bs_f = pl.BlockSpec((1, _G), lambda i: (0, 0))
    return pl.pallas_call(
        _tcf_body,
        grid=(_NPAD // _R,),
        in_specs=[bs_f, bs_g, bs_g, bs_c, bs_c, bs_r, bs_w, bs_b, bs_w],
        out_specs=pl.BlockSpec((_G, _D), lambda i: (0, 0)),
        out_shape=jax.ShapeDtypeStruct((_G, _D), _F32),
        scratch_shapes=[pltpu.VMEM((_G, _D), _F32)],
    )(f, s0, s1, c0, c1, h, wlT, b, wrT)


def kernel(x, edge_index, batch, W_l0, b_l0, W_r0, W_l1, b_l1, W_r1):
    src = edge_index[0]
    dst = edge_index[1]
    padlen = _NW * _EPW - _E
    srcp = jnp.concatenate([src, jnp.zeros((padlen,), jnp.int32)]
                           ).reshape(_NW, _NCH, _CH)
    dstp = jnp.concatenate([dst, jnp.full((padlen,), _N, jnp.int32)]
                           ).reshape(_NW, _NCH, _CH)
    ones = jnp.ones((_CH,), _F32)
    zr = jnp.zeros((_RPT, _D), _F32)
    zc = jnp.zeros((_RPT,), _F32)
    xpad = jnp.pad(x, ((0, _NPAD - _N), (0, 0)))
    batchp = jnp.pad(batch, (0, _NPAD - _N),
                     constant_values=_G - 1).reshape(_NPAD, 1)

    sum1, cnt1 = _sc_agg(xpad, srcp, dstp, ones, zr, zc)
    h = _tc1(sum1[0], sum1[1],
             cnt1[0].reshape(_NPAD, 1), cnt1[1].reshape(_NPAD, 1),
             xpad, W_l0.T, b_l0.reshape(1, _D), W_r0.T)
    f, slot = _firsts(batchp)
    sum2, cnt2 = _sc_agg2(h, srcp.reshape(_NW, _EPW), dstp.reshape(_NW, _EPW),
                          slot.reshape(_NPAD), ones, zr, zc)
    return _tcf(f, sum2[0, :_G], sum2[1, :_G],
                cnt2[0, :_G].reshape(_G, 1), cnt2[1, :_G].reshape(_G, 1),
                h, W_l1.T, b_l1.reshape(1, _D), W_r1.T)


# speedup vs baseline: 5.7590x; 1.1126x over previous
import jax
import jax.numpy as jnp
from jax import lax
from jax.experimental import pallas as pl
from jax.experimental.pallas import tpu as pltpu
from jax.experimental.pallas import tpu_sc as plsc

_N = 10000
_E = 320000
_D = 128
_G = 64
_NPAD = 10240
_NSUB = 16
_NCORE = 2
_NW = _NCORE * _NSUB
_CH = 128
_EPW = 10240
_NCH = _EPW // _CH
_RPT = _NPAD // _NSUB

_F32 = jnp.float32


def _sc_agg_body(feat_hbm, srcp_hbm, dstp_hbm, ones_hbm, zr_hbm, zc_hbm,
                 sum_out, cnt_out,
                 src_all, dst_all, rbuf0, rbuf1, ones_v, gsem0, gsem1,
                 acc, cntacc):
    c = lax.axis_index("c")
    s = lax.axis_index("s")
    wid = c * _NSUB + s
    pltpu.sync_copy(ones_hbm, ones_v)
    base = s * _RPT
    pltpu.sync_copy(zr_hbm, acc.at[pl.ds(base, _RPT)])
    pltpu.sync_copy(zc_hbm, cntacc.at[pl.ds(base, _RPT)])
    plsc.subcore_barrier()

    nhc = _NCH // 2

    def half(hh, carry):
        pltpu.sync_copy(srcp_hbm.at[wid, pl.ds(hh * nhc, nhc)], src_all)
        pltpu.sync_copy(dstp_hbm.at[wid, pl.ds(hh * nhc, nhc)], dst_all)
        pltpu.async_copy(feat_hbm.at[src_all.at[0]], rbuf0, gsem0)
        pltpu.async_copy(feat_hbm.at[src_all.at[1]], rbuf1, gsem1)

        def chunk2(t, carry2):
            j0 = 2 * t
            j1 = j0 + 1
            pltpu.make_async_copy(feat_hbm.at[src_all.at[j0]],
                                  rbuf0, gsem0).wait()
            pltpu.sync_copy(rbuf0, acc.at[dst_all.at[j0]], add=True)
            pltpu.sync_copy(ones_v, cntacc.at[dst_all.at[j0]], add=True)

            @pl.when(t < nhc // 2 - 1)
            def _p0():
                pltpu.async_copy(feat_hbm.at[src_all.at[j0 + 2]], rbuf0, gsem0)

            pltpu.make_async_copy(feat_hbm.at[src_all.at[j1]],
                                  rbuf1, gsem1).wait()
            pltpu.sync_copy(rbuf1, acc.at[dst_all.at[j1]], add=True)
            pltpu.sync_copy(ones_v, cntacc.at[dst_all.at[j1]], add=True)

            @pl.when(t < nhc // 2 - 1)
            def _p1():
                pltpu.async_copy(feat_hbm.at[src_all.at[j1 + 2]], rbuf1, gsem1)

            return carry2

        lax.fori_loop(0, nhc // 2, chunk2, 0)
        return carry

    lax.fori_loop(0, 2, half, 0)
    plsc.subcore_barrier()
    pltpu.sync_copy(acc.at[pl.ds(base, _RPT)], sum_out.at[c, pl.ds(base, _RPT)])
    pltpu.sync_copy(cntacc.at[pl.ds(base, _RPT)], cnt_out.at[c, pl.ds(base, _RPT)])


_SL = 128


def _sc_agg2_body(feat_hbm, srci_hbm, dsti_hbm, slot_hbm, ones_hbm,
                  zr_hbm, zc_hbm,
                  sum_out, cnt_out,
                  srcf, dstf, slottab, fsrcf, fslotf, sidx, rbuf, ones_v,
                  gsem, acc, cntacc):
    c = lax.axis_index("c")
    s = lax.axis_index("s")
    wid = c * _NSUB + s
    pltpu.sync_copy(srci_hbm.at[wid], srcf)
    pltpu.sync_copy(dsti_hbm.at[wid], dstf)
    pltpu.sync_copy(slot_hbm, slottab)
    pltpu.sync_copy(ones_hbm, ones_v)

    @pl.when(s == 0)
    def _zero():
        pltpu.sync_copy(zr_hbm.at[pl.ds(0, _SL)], acc)
        pltpu.sync_copy(zc_hbm.at[pl.ds(0, _SL)], cntacc)

    def pre(i, carry):
        fsrcf[pl.ds(i * 16, 16)] = jnp.zeros((16,), jnp.int32)
        fslotf[pl.ds(i * 16, 16)] = jnp.full((16,), _G, jnp.int32)
        return carry

    lax.fori_loop(0, (_EPW + 16) // 16, pre, 0)
    plsc.subcore_barrier()

    def scan(i, fcnt):
        dv = dstf[pl.ds(i * 16, 16)]
        sl = plsc.load_gather(slottab, [dv])
        m = sl >= 0
        sv = srcf[pl.ds(i * 16, 16)]
        plsc.store_compressed(fsrcf.at[pl.ds(fcnt, 16)], sv, mask=m)
        plsc.store_compressed(fslotf.at[pl.ds(fcnt, 16)], sl, mask=m)
        return fcnt + jnp.sum(m.astype(jnp.int32))

    fcnt = lax.fori_loop(0, _EPW // 16, scan, 0)
    nch = (fcnt + _CH - 1) // _CH

    def chunk(j, carry):
        for k in range(8):
            sidx[pl.ds(k * 16, 16)] = fslotf[pl.ds(j * _CH + k * 16, 16)]
        pltpu.async_copy(feat_hbm.at[fsrcf.at[pl.ds(j * _CH, _CH)]],
                         rbuf, gsem).wait()
        pltpu.sync_copy(rbuf, acc.at[sidx], add=True)
        pltpu.sync_copy(ones_v, cntacc.at[sidx], add=True)
        return carry

    lax.fori_loop(0, nch, chunk, 0)
    plsc.subcore_barrier()

    @pl.when(s == 0)
    def _write():
        pltpu.sync_copy(acc, sum_out.at[c])
        pltpu.sync_copy(cntacc, cnt_out.at[c])


_sc_agg2 = pl.kernel(
    _sc_agg2_body,
    out_type=[jax.ShapeDtypeStruct((_NCORE, _SL, _D), _F32),
              jax.ShapeDtypeStruct((_NCORE, _SL), _F32)],
    mesh=plsc.VectorSubcoreMesh(core_axis_name="c", subcore_axis_name="s"),
    scratch_types=[
        pltpu.VMEM((_EPW,), jnp.int32),
        pltpu.VMEM((_EPW,), jnp.int32),
        pltpu.VMEM((_NPAD,), jnp.int32),
        pltpu.VMEM((_EPW + 16,), jnp.int32),
        pltpu.VMEM((_EPW + 16,), jnp.int32),
        pltpu.VMEM((_CH,), jnp.int32),
        pltpu.VMEM((_CH, _D), _F32),
        pltpu.VMEM((_CH,), _F32),
        pltpu.SemaphoreType.DMA,
        pltpu.VMEM_SHARED((_SL, _D), _F32),
        pltpu.VMEM_SHARED((_SL,), _F32),
    ],
    compiler_params=pltpu.CompilerParams(needs_layout_passes=False),
)


_sc_agg = pl.kernel(
    _sc_agg_body,
    out_type=[jax.ShapeDtypeStruct((_NCORE, _NPAD, _D), _F32),
              jax.ShapeDtypeStruct((_NCORE, _NPAD), _F32)],
    mesh=plsc.VectorSubcoreMesh(core_axis_name="c", subcore_axis_name="s"),
    scratch_types=[
        pltpu.VMEM((_NCH // 2, _CH), jnp.int32),
        pltpu.VMEM((_NCH // 2, _CH), jnp.int32),
        pltpu.VMEM((_CH, _D), _F32),
        pltpu.VMEM((_CH, _D), _F32),
        pltpu.VMEM((_CH,), _F32),
        pltpu.SemaphoreType.DMA,
        pltpu.SemaphoreType.DMA,
        pltpu.VMEM_SHARED((_NPAD, _D), _F32),
        pltpu.VMEM_SHARED((_NPAD,), _F32),
    ],
    compiler_params=pltpu.CompilerParams(needs_layout_passes=False),
)


_R = 1024

def _dot(a, b):
    return jnp.dot(a, b, preferred_element_type=_F32,
                   precision=lax.Precision.HIGHEST)


def _tc1_body(s0, s1, c0, c1, x, wl, bb, wr, o):
    cnt = jnp.maximum(c0[...] + c1[...], 1.0)
    mean = (s0[...] + s1[...]) / cnt
    h = _dot(mean, wl[...]) + bb[...] + _dot(x[...], wr[...])
    o[...] = jnp.maximum(h, 0.0)


def _tc1(s0, s1, c0, c1, xpad, wlT, b, wrT):
    bs_r = pl.BlockSpec((_R, _D), lambda i: (i, 0))
    bs_c = pl.BlockSpec((_R, 1), lambda i: (i, 0))
    bs_w = pl.BlockSpec((_D, _D), lambda i: (0, 0))
    bs_b = pl.BlockSpec((1, _D), lambda i: (0, 0))
    return pl.pallas_call(
        _tc1_body,
        grid=(_NPAD // _R,),
        in_specs=[bs_r, bs_r, bs_c, bs_c, bs_r, bs_w, bs_b, bs_w],
        out_specs=bs_r,
        out_shape=jax.ShapeDtypeStruct((_NPAD, _D), _F32),
    )(s0, s1, c0, c1, xpad, wlT, b, wrT)


def _firsts_body(b_ref, f_ref, slot_ref):
    g = lax.broadcasted_iota(jnp.int32, (1, _G), 1)
    lt = (b_ref[...] < g).astype(jnp.int32)
    f = jnp.minimum(jnp.sum(lt, axis=0, keepdims=True), _N - 1)
    f_ref[...] = f
    rows = lax.broadcasted_iota(jnp.int32, (_NPAD, 1), 0)
    cand = jnp.where(rows == f, g, _G)
    sl = jnp.min(cand, axis=1, keepdims=True)
    slot_ref[...] = jnp.where(sl == _G, -1, sl)


def _firsts(batchp):
    return pl.pallas_call(
        _firsts_body,
        grid=(1,),
        in_specs=[pl.BlockSpec((_NPAD, 1), lambda i: (0, 0))],
        out_specs=[pl.BlockSpec((1, _G), lambda i: (0, 0)),
                   pl.BlockSpec((_NPAD, 1), lambda i: (0, 0))],
        out_shape=[jax.ShapeDtypeStruct((1, _G), jnp.int32),
                   jax.ShapeDtypeStruct((_NPAD, 1), jnp.int32)],
    )(batchp)


def _tcf_body(f, s0, s1, c0, c1, h, wl, bb, wr, o, acch):
    i = pl.program_id(0)

    @pl.when(i == 0)
    def _init():
        acch[...] = jnp.zeros_like(acch)

    rows = lax.broadcasted_iota(jnp.int32, (_R, 1), 0) + i * _R
    oh = (rows == f[...]).astype(_F32)
    acch[...] += lax.dot_general(oh, h[...], (((0,), (0,)), ((), ())),
                                 preferred_element_type=_F32,
                                 precision=lax.Precision.HIGHEST)

    @pl.when(i == _NPAD // _R - 1)
    def _fin():
        fv = f[...]
        gp = lax.broadcasted_iota(jnp.int32, (_G, _G), 1)
        eqm = jnp.reshape(fv, (_G, 1)) == fv
        dmin = jnp.min(jnp.where(eqm, gp, _G), axis=1, keepdims=True)
        P = (gp == dmin).astype(_F32)
        cnt = jnp.maximum(c0[...] + c1[...], 1.0)
        mean = (s0[...] + s1[...]) / cnt
        mean_sel = _dot(P, mean)
        o[...] = _dot(mean_sel, wl[...]) + bb[...] + _dot(acch[...], wr[...])


def _tcf(f, s0, s1, c0, c1, h, wlT, b, wrT):
    bs_r = pl.BlockSpec((_R, _D), lambda i: (i, 0))
    bs_g = pl.BlockSpec((_G, _D), lambda i: (0, 0))
    bs_c = pl.BlockSpec((_G, 1), lambda i: (0, 0))
    bs_w = pl.BlockSpec((_D, _D), lambda i: (0, 0))
    bs_b = pl.BlockSpec((1, _D), lambda i: (0, 0))
    bs_f = pl.BlockSpec((1, _G), lambda i: (0, 0))
    return pl.pallas_call(
        _tcf_body,
        grid=(_NPAD // _R,),
        in_specs=[bs_f, bs_g, bs_g, bs_c, bs_c, bs_r, bs_w, bs_b, bs_w],
        out_specs=pl.BlockSpec((_G, _D), lambda i: (0, 0)),
        out_shape=jax.ShapeDtypeStruct((_G, _D), _F32),
        scratch_shapes=[pltpu.VMEM((_G, _D), _F32)],
    )(f, s0, s1, c0, c1, h, wlT, b, wrT)


def kernel(x, edge_index, batch, W_l0, b_l0, W_r0, W_l1, b_l1, W_r1):
    src = edge_index[0]
    dst = edge_index[1]
    padlen = _NW * _EPW - _E
    srcp = jnp.concatenate([src, jnp.zeros((padlen,), jnp.int32)]
                           ).reshape(_NW, _NCH, _CH)
    dstp = jnp.concatenate([dst, jnp.full((padlen,), _N, jnp.int32)]
                           ).reshape(_NW, _NCH, _CH)
    ones = jnp.ones((_CH,), _F32)
    zr = jnp.zeros((_RPT, _D), _F32)
    zc = jnp.zeros((_RPT,), _F32)
    xpad = jnp.pad(x, ((0, _NPAD - _N), (0, 0)))
    batchp = jnp.pad(batch, (0, _NPAD - _N),
                     constant_values=_G - 1).reshape(_NPAD, 1)

    sum1, cnt1 = _sc_agg(xpad, srcp, dstp, ones, zr, zc)
    h = _tc1(sum1[0], sum1[1],
             cnt1[0].reshape(_NPAD, 1), cnt1[1].reshape(_NPAD, 1),
             xpad, W_l0.T, b_l0.reshape(1, _D), W_r0.T)
    f, slot = _firsts(batchp)
    sum2, cnt2 = _sc_agg2(h, srcp.reshape(_NW, _EPW), dstp.reshape(_NW, _EPW),
                          slot.reshape(_NPAD), ones, zr, zc)
    return _tcf(f, sum2[0, :_G], sum2[1, :_G],
                cnt2[0, :_G].reshape(_G, 1), cnt2[1, :_G].reshape(_G, 1),
                h, W_l1.T, b_l1.reshape(1, _D), W_r1.T)


# speedup vs baseline: 11.1262x; 1.9320x over previous
import jax
import jax.numpy as jnp
from jax import lax
from jax.experimental import pallas as pl
from jax.experimental.pallas import tpu as pltpu
from jax.experimental.pallas import tpu_sc as plsc

_N = 10000
_E = 320000
_D = 128
_G = 64
_NPAD = 10240
_NSUB = 16
_NCORE = 2
_NW = _NCORE * _NSUB
_CH = 128
_EPW = 10240
_NCH = _EPW // _CH
_RPT = _NPAD // _NSUB

_F32 = jnp.float32


def _sc_agg_body(feat_hbm, srcp_hbm, dstp_hbm, ones_hbm, zr_hbm, zc_hbm,
                 sum_out, cnt_out,
                 src_all, dst_all, rbuf0, rbuf1, ones_v, gsem0, gsem1,
                 acc, cntacc):
    c = lax.axis_index("c")
    s = lax.axis_index("s")
    wid = c * _NSUB + s
    pltpu.sync_copy(ones_hbm, ones_v)
    base = s * _RPT
    pltpu.sync_copy(zr_hbm, acc.at[pl.ds(base, _RPT)])
    pltpu.sync_copy(zc_hbm, cntacc.at[pl.ds(base, _RPT)])
    plsc.subcore_barrier()

    nhc = _NCH // 2

    def half(hh, carry):
        pltpu.sync_copy(srcp_hbm.at[wid, pl.ds(hh * nhc, nhc)], src_all)
        pltpu.sync_copy(dstp_hbm.at[wid, pl.ds(hh * nhc, nhc)], dst_all)
        pltpu.async_copy(feat_hbm.at[src_all.at[0]], rbuf0, gsem0)
        pltpu.async_copy(feat_hbm.at[src_all.at[1]], rbuf1, gsem1)

        def chunk2(t, carry2):
            j0 = 2 * t
            j1 = j0 + 1
            pltpu.make_async_copy(feat_hbm.at[src_all.at[j0]],
                                  rbuf0, gsem0).wait()
            pltpu.sync_copy(rbuf0, acc.at[dst_all.at[j0]], add=True)
            pltpu.sync_copy(ones_v, cntacc.at[dst_all.at[j0]], add=True)

            @pl.when(t < nhc // 2 - 1)
            def _p0():
                pltpu.async_copy(feat_hbm.at[src_all.at[j0 + 2]], rbuf0, gsem0)

            pltpu.make_async_copy(feat_hbm.at[src_all.at[j1]],
                                  rbuf1, gsem1).wait()
            pltpu.sync_copy(rbuf1, acc.at[dst_all.at[j1]], add=True)
            pltpu.sync_copy(ones_v, cntacc.at[dst_all.at[j1]], add=True)

            @pl.when(t < nhc // 2 - 1)
            def _p1():
                pltpu.async_copy(feat_hbm.at[src_all.at[j1 + 2]], rbuf1, gsem1)

            return carry2

        lax.fori_loop(0, nhc // 2, chunk2, 0)
        return carry

    lax.fori_loop(0, 2, half, 0)
    plsc.subcore_barrier()
    pltpu.sync_copy(acc.at[pl.ds(base, _RPT)], sum_out.at[c, pl.ds(base, _RPT)])
    pltpu.sync_copy(cntacc.at[pl.ds(base, _RPT)], cnt_out.at[c, pl.ds(base, _RPT)])


_SL = 128
_CB = 10496
_MTRASH = 10016
_CHR = 64


def _sc_filter_body(srci_hbm, dsti_hbm, slot_hbm, ones_hbm, zc_hbm,
                    mark_out,
                    srcf, dstf, slottab, sidx, ones_v, markacc):
    c = lax.axis_index("c")
    s = lax.axis_index("s")
    wid = c * _NSUB + s
    pltpu.sync_copy(srci_hbm.at[wid], srcf.at[pl.ds(0, _EPW)])
    pltpu.sync_copy(dsti_hbm.at[wid], dstf)
    pltpu.sync_copy(slot_hbm, slottab)
    pltpu.sync_copy(ones_hbm, ones_v)
    base = s * _RPT
    pltpu.sync_copy(zc_hbm, markacc.at[pl.ds(base, _RPT)])
    plsc.subcore_barrier()

    def scan(i, fcnt):
        dv = dstf[pl.ds(i * 16, 16)]
        sl = plsc.load_gather(slottab, [dv])
        m = sl >= 0
        sv = srcf[pl.ds(i * 16, 16)]
        plsc.store_compressed(srcf.at[pl.ds(fcnt, 16)], sv, mask=m)
        return fcnt + jnp.sum(m.astype(jnp.int32))

    fcnt = lax.fori_loop(0, _EPW // 16, scan, 0)

    def appf(i, fcnt):
        sl = slottab[pl.ds(base + i * 16, 16)]
        m = sl >= 0
        ids = jnp.full((16,), base + i * 16, jnp.int32) + lax.iota(jnp.int32, 16)
        plsc.store_compressed(srcf.at[pl.ds(fcnt, 16)], ids, mask=m)
        return fcnt + jnp.sum(m.astype(jnp.int32))

    fcnt = lax.fori_loop(0, _RPT // 16, appf, fcnt)

    off = (fcnt // 16) * 16
    for k in range(_CH // 16 + 1):
        lanes = jnp.full((16,), off + k * 16, jnp.int32) + lax.iota(jnp.int32, 16)
        cur = srcf[pl.ds(off + k * 16, 16)]
        srcf[pl.ds(off + k * 16, 16)] = jnp.where(lanes >= fcnt, _MTRASH, cur)

    nch = (fcnt + _CH - 1) // _CH

    def chunk(j, carry):
        for k in range(_CH // 16):
            sidx[pl.ds(k * 16, 16)] = srcf[pl.ds(j * _CH + k * 16, 16)]
        pltpu.sync_copy(ones_v, markacc.at[sidx], add=True)
        return carry

    lax.fori_loop(0, nch, chunk, 0)
    plsc.subcore_barrier()
    pltpu.sync_copy(markacc.at[pl.ds(base, _RPT)],
                    mark_out.at[c, pl.ds(base, _RPT)])


_sc_filter = pl.kernel(
    _sc_filter_body,
    out_type=[jax.ShapeDtypeStruct((_NCORE, _NPAD), _F32)],
    mesh=plsc.VectorSubcoreMesh(core_axis_name="c", subcore_axis_name="s"),
    scratch_types=[
        pltpu.VMEM((_CB,), jnp.int32),
        pltpu.VMEM((_EPW,), jnp.int32),
        pltpu.VMEM((_NPAD,), jnp.int32),
        pltpu.VMEM((_CH,), jnp.int32),
        pltpu.VMEM((_CH,), _F32),
        pltpu.VMEM_SHARED((_NPAD,), _F32),
    ],
    compiler_params=pltpu.CompilerParams(needs_layout_passes=False),
)


def _sc_restricted_body(feat_hbm, srci_hbm, dsti_hbm, m0_hbm, m1_hbm,
                        ones_hbm, zr_hbm, zc_hbm,
                        sum_out, cnt_out,
                        srcf, dstf, needtab, m1buf, sidx0, sidx1, ones_v,
                        rbuf0, rbuf1, gsem0, gsem1, acc, cntacc):
    c = lax.axis_index("c")
    s = lax.axis_index("s")
    wid = c * _NSUB + s
    pltpu.sync_copy(srci_hbm.at[wid], srcf.at[pl.ds(0, _EPW)])
    pltpu.sync_copy(dsti_hbm.at[wid], dstf.at[pl.ds(0, _EPW)])
    pltpu.sync_copy(m0_hbm, needtab)
    pltpu.sync_copy(ones_hbm, ones_v)
    base = s * _RPT
    pltpu.sync_copy(zr_hbm, acc.at[pl.ds(base, _RPT)])
    pltpu.sync_copy(zc_hbm, cntacc.at[pl.ds(base, _RPT)])

    def comb(b, carry):
        pltpu.sync_copy(m1_hbm.at[pl.ds(b * 512, 512)], m1buf)

        def add16(i, carry2):
            o = b * 512 + i * 16
            needtab[pl.ds(o, 16)] = (needtab[pl.ds(o, 16)]
                                     + m1buf[pl.ds(i * 16, 16)])
            return carry2

        return lax.fori_loop(0, 32, add16, carry)

    lax.fori_loop(0, _NPAD // 512, comb, 0)
    plsc.subcore_barrier()

    def scan(i, fcnt):
        dv = dstf[pl.ds(i * 16, 16)]
        nv = plsc.load_gather(needtab, [dv])
        m = nv > 0.0
        sv = srcf[pl.ds(i * 16, 16)]
        plsc.store_compressed(srcf.at[pl.ds(fcnt, 16)], sv, mask=m)
        plsc.store_compressed(dstf.at[pl.ds(fcnt, 16)], dv, mask=m)
        return fcnt + jnp.sum(m.astype(jnp.int32))

    fcnt = lax.fori_loop(0, _EPW // 16, scan, 0)

    off = (fcnt // 16) * 16
    for k in range(_CHR // 16 + 1):
        lanes = jnp.full((16,), off + k * 16, jnp.int32) + lax.iota(jnp.int32, 16)
        mpad = lanes >= fcnt
        cs = srcf[pl.ds(off + k * 16, 16)]
        srcf[pl.ds(off + k * 16, 16)] = jnp.where(mpad, 0, cs)
        cd = dstf[pl.ds(off + k * 16, 16)]
        dstf[pl.ds(off + k * 16, 16)] = jnp.where(mpad, _N, cd)

    nch = (fcnt + _CHR - 1) // _CHR

    @pl.when(nch > 0)
    def _g0():
        pltpu.async_copy(feat_hbm.at[srcf.at[pl.ds(0, _CHR)]], rbuf0, gsem0)

    @pl.when(nch > 1)
    def _g1():
        pltpu.async_copy(feat_hbm.at[srcf.at[pl.ds(_CHR, _CHR)]], rbuf1, gsem1)

    def chunk2(t, carry):
        j0 = 2 * t
        j1 = j0 + 1
        pltpu.make_async_copy(feat_hbm.at[srcf.at[pl.ds(j0 * _CHR, _CHR)]],
                              rbuf0, gsem0).wait()
        for k in range(_CHR // 16):
            sidx0[pl.ds(k * 16, 16)] = dstf[pl.ds(j0 * _CHR + k * 16, 16)]
        pltpu.sync_copy(rbuf0, acc.at[sidx0], add=True)
        pltpu.sync_copy(ones_v.at[pl.ds(0, _CHR)], cntacc.at[sidx0], add=True)

        @pl.when(j0 + 2 < nch)
        def _p0():
            pltpu.async_copy(feat_hbm.at[srcf.at[pl.ds((j0 + 2) * _CHR, _CHR)]],
                             rbuf0, gsem0)

        @pl.when(j1 < nch)
        def _odd():
            pltpu.make_async_copy(feat_hbm.at[srcf.at[pl.ds(j1 * _CHR, _CHR)]],
                                  rbuf1, gsem1).wait()
            for k in range(_CHR // 16):
                sidx1[pl.ds(k * 16, 16)] = dstf[pl.ds(j1 * _CHR + k * 16, 16)]
            pltpu.sync_copy(rbuf1, acc.at[sidx1], add=True)
            pltpu.sync_copy(ones_v.at[pl.ds(0, _CHR)], cntacc.at[sidx1],
                            add=True)

            @pl.when(j1 + 2 < nch)
            def _p1():
                pltpu.async_copy(
                    feat_hbm.at[srcf.at[pl.ds((j1 + 2) * _CHR, _CHR)]],
                    rbuf1, gsem1)

        return carry

    lax.fori_loop(0, (nch + 1) // 2, chunk2, 0)
    plsc.subcore_barrier()
    pltpu.sync_copy(acc.at[pl.ds(base, _RPT)], sum_out.at[c, pl.ds(base, _RPT)])
    pltpu.sync_copy(cntacc.at[pl.ds(base, _RPT)],
                    cnt_out.at[c, pl.ds(base, _RPT)])


_sc_restricted = pl.kernel(
    _sc_restricted_body,
    out_type=[jax.ShapeDtypeStruct((_NCORE, _NPAD, _D), _F32),
              jax.ShapeDtypeStruct((_NCORE, _NPAD), _F32)],
    mesh=plsc.VectorSubcoreMesh(core_axis_name="c", subcore_axis_name="s"),
    scratch_types=[
        pltpu.VMEM((_CB,), jnp.int32),
        pltpu.VMEM((_CB,), jnp.int32),
        pltpu.VMEM((_NPAD,), _F32),
        pltpu.VMEM((512,), _F32),
        pltpu.VMEM((_CHR,), jnp.int32),
        pltpu.VMEM((_CHR,), jnp.int32),
        pltpu.VMEM((_CH,), _F32),
        pltpu.VMEM((_CHR, _D), _F32),
        pltpu.VMEM((_CHR, _D), _F32),
        pltpu.SemaphoreType.DMA,
        pltpu.SemaphoreType.DMA,
        pltpu.VMEM_SHARED((_NPAD, _D), _F32),
        pltpu.VMEM_SHARED((_NPAD,), _F32),
    ],
    compiler_params=pltpu.CompilerParams(needs_layout_passes=False),
)


def _sc_agg2_body(feat_hbm, srci_hbm, dsti_hbm, slot_hbm, ones_hbm,
                  zr_hbm, zc_hbm,
                  sum_out, cnt_out,
                  srcf, dstf, slottab, fsrcf, fslotf, sidx, rbuf, ones_v,
                  gsem, acc, cntacc):
    c = lax.axis_index("c")
    s = lax.axis_index("s")
    wid = c * _NSUB + s
    pltpu.sync_copy(srci_hbm.at[wid], srcf)
    pltpu.sync_copy(dsti_hbm.at[wid], dstf)
    pltpu.sync_copy(slot_hbm, slottab)
    pltpu.sync_copy(ones_hbm, ones_v)

    @pl.when(s == 0)
    def _zero():
        pltpu.sync_copy(zr_hbm.at[pl.ds(0, _SL)], acc)
        pltpu.sync_copy(zc_hbm.at[pl.ds(0, _SL)], cntacc)

    def pre(i, carry):
        fsrcf[pl.ds(i * 16, 16)] = jnp.zeros((16,), jnp.int32)
        fslotf[pl.ds(i * 16, 16)] = jnp.full((16,), _G, jnp.int32)
        return carry

    lax.fori_loop(0, (_EPW + 16) // 16, pre, 0)
    plsc.subcore_barrier()

    def scan(i, fcnt):
        dv = dstf[pl.ds(i * 16, 16)]
        sl = plsc.load_gather(slottab, [dv])
        m = sl >= 0
        sv = srcf[pl.ds(i * 16, 16)]
        plsc.store_compressed(fsrcf.at[pl.ds(fcnt, 16)], sv, mask=m)
        plsc.store_compressed(fslotf.at[pl.ds(fcnt, 16)], sl, mask=m)
        return fcnt + jnp.sum(m.astype(jnp.int32))

    fcnt = lax.fori_loop(0, _EPW // 16, scan, 0)
    nch = (fcnt + _CH - 1) // _CH

    def chunk(j, carry):
        for k in range(8):
            sidx[pl.ds(k * 16, 16)] = fslotf[pl.ds(j * _CH + k * 16, 16)]
        pltpu.async_copy(feat_hbm.at[fsrcf.at[pl.ds(j * _CH, _CH)]],
                         rbuf, gsem).wait()
        pltpu.sync_copy(rbuf, acc.at[sidx], add=True)
        pltpu.sync_copy(ones_v, cntacc.at[sidx], add=True)
        return carry

    lax.fori_loop(0, nch, chunk, 0)
    plsc.subcore_barrier()

    @pl.when(s == 0)
    def _write():
        pltpu.sync_copy(acc, sum_out.at[c])
        pltpu.sync_copy(cntacc, cnt_out.at[c])


_sc_agg2 = pl.kernel(
    _sc_agg2_body,
    out_type=[jax.ShapeDtypeStruct((_NCORE, _SL, _D), _F32),
              jax.ShapeDtypeStruct((_NCORE, _SL), _F32)],
    mesh=plsc.VectorSubcoreMesh(core_axis_name="c", subcore_axis_name="s"),
    scratch_types=[
        pltpu.VMEM((_EPW,), jnp.int32),
        pltpu.VMEM((_EPW,), jnp.int32),
        pltpu.VMEM((_NPAD,), jnp.int32),
        pltpu.VMEM((_EPW + 16,), jnp.int32),
        pltpu.VMEM((_EPW + 16,), jnp.int32),
        pltpu.VMEM((_CH,), jnp.int32),
        pltpu.VMEM((_CH, _D), _F32),
        pltpu.VMEM((_CH,), _F32),
        pltpu.SemaphoreType.DMA,
        pltpu.VMEM_SHARED((_SL, _D), _F32),
        pltpu.VMEM_SHARED((_SL,), _F32),
    ],
    compiler_params=pltpu.CompilerParams(needs_layout_passes=False),
)


_sc_agg = pl.kernel(
    _sc_agg_body,
    out_type=[jax.ShapeDtypeStruct((_NCORE, _NPAD, _D), _F32),
              jax.ShapeDtypeStruct((_NCORE, _NPAD), _F32)],
    mesh=plsc.VectorSubcoreMesh(core_axis_name="c", subcore_axis_name="s"),
    scratch_types=[
        pltpu.VMEM((_NCH // 2, _CH), jnp.int32),
        pltpu.VMEM((_NCH // 2, _CH), jnp.int32),
        pltpu.VMEM((_CH, _D), _F32),
        pltpu.VMEM((_CH, _D), _F32),
        pltpu.VMEM((_CH,), _F32),
        pltpu.SemaphoreType.DMA,
        pltpu.SemaphoreType.DMA,
        pltpu.VMEM_SHARED((_NPAD, _D), _F32),
        pltpu.VMEM_SHARED((_NPAD,), _F32),
    ],
    compiler_params=pltpu.CompilerParams(needs_layout_passes=False),
)


_R = 1024

def _dot(a, b):
    return jnp.dot(a, b, preferred_element_type=_F32,
                   precision=lax.Precision.HIGHEST)


def _tc1_body(s0, s1, c0, c1, x, wl, bb, wr, o):
    cnt = jnp.maximum(c0[...] + c1[...], 1.0)
    mean = (s0[...] + s1[...]) / cnt
    h = _dot(mean, wl[...]) + bb[...] + _dot(x[...], wr[...])
    o[...] = jnp.maximum(h, 0.0)


def _tc1(s0, s1, c0, c1, xpad, wlT, b, wrT):
    bs_r = pl.BlockSpec((_R, _D), lambda i: (i, 0))
    bs_c = pl.BlockSpec((_R, 1), lambda i: (i, 0))
    bs_w = pl.BlockSpec((_D, _D), lambda i: (0, 0))
    bs_b = pl.BlockSpec((1, _D), lambda i: (0, 0))
    return pl.pallas_call(
        _tc1_body,
        grid=(_NPAD // _R,),
        in_specs=[bs_r, bs_r, bs_c, bs_c, bs_r, bs_w, bs_b, bs_w],
        out_specs=bs_r,
        out_shape=jax.ShapeDtypeStruct((_NPAD, _D), _F32),
    )(s0, s1, c0, c1, xpad, wlT, b, wrT)


def _firsts_body(b_ref, f_ref, slot_ref):
    g = lax.broadcasted_iota(jnp.int32, (1, _G), 1)
    lt = (b_ref[...] < g).astype(jnp.int32)
    f = jnp.minimum(jnp.sum(lt, axis=0, keepdims=True), _N - 1)
    f_ref[...] = f
    rows = lax.broadcasted_iota(jnp.int32, (_NPAD, 1), 0)
    cand = jnp.where(rows == f, g, _G)
    sl = jnp.min(cand, axis=1, keepdims=True)
    slot_ref[...] = jnp.where(sl == _G, -1, sl)


def _firsts(batchp):
    return pl.pallas_call(
        _firsts_body,
        grid=(1,),
        in_specs=[pl.BlockSpec((_NPAD, 1), lambda i: (0, 0))],
        out_specs=[pl.BlockSpec((1, _G), lambda i: (0, 0)),
                   pl.BlockSpec((_NPAD, 1), lambda i: (0, 0))],
        out_shape=[jax.ShapeDtypeStruct((1, _G), jnp.int32),
                   jax.ShapeDtypeStruct((_NPAD, 1), jnp.int32)],
    )(batchp)


def _tcf_body(f, s0, s1, c0, c1, h, wl, bb, wr, o, acch):
    i = pl.program_id(0)

    @pl.when(i == 0)
    def _init():
        acch[...] = jnp.zeros_like(acch)

    rows = lax.broadcasted_iota(jnp.int32, (_R, 1), 0) + i * _R
    oh = (rows == f[...]).astype(_F32)
    acch[...] += lax.dot_general(oh, h[...], (((0,), (0,)), ((), ())),
                                 preferred_element_type=_F32,
                                 precision=lax.Precision.HIGHEST)

    @pl.when(i == _NPAD // _R - 1)
    def _fin():
        fv = f[...]
        gp = lax.broadcasted_iota(jnp.int32, (_G, _G), 1)
        eqm = jnp.reshape(fv, (_G, 1)) == fv
        dmin = jnp.min(jnp.where(eqm, gp, _G), axis=1, keepdims=True)
        P = (gp == dmin).astype(_F32)
        cnt = jnp.maximum(c0[...] + c1[...], 1.0)
        mean = (s0[...] + s1[...]) / cnt
        mean_sel = _dot(P, mean)
        o[...] = _dot(mean_sel, wl[...]) + bb[...] + _dot(acch[...], wr[...])


def _tcf(f, s0, s1, c0, c1, h, wlT, b, wrT):
    bs_r = pl.BlockSpec((_R, _D), lambda i: (i, 0))
    bs_g = pl.BlockSpec((_G, _D), lambda i: (0, 0))
    bs_c = pl.BlockSpec((_G, 1), lambda i: (0, 0))
    bs_w = pl.BlockSpec((_D, _D), lambda i: (0, 0))
    bs_b = pl.BlockSpec((1, _D), lambda i: (0, 0))
    bs_f = pl.BlockSpec((1, _G), lambda i: (0, 0))
    return pl.pallas_call(
        _tcf_body,
        grid=(_NPAD // _R,),
        in_specs=[bs_f, bs_g, bs_g, bs_c, bs_c, bs_r, bs_w, bs_b, bs_w],
        out_specs=pl.BlockSpec((_G, _D), lambda i: (0, 0)),
        out_shape=jax.ShapeDtypeStruct((_G, _D), _F32),
        scratch_shapes=[pltpu.VMEM((_G, _D), _F32)],
    )(f, s0, s1, c0, c1, h, wlT, b, wrT)


def kernel(x, edge_index, batch, W_l0, b_l0, W_r0, W_l1, b_l1, W_r1):
    src = edge_index[0]
    dst = edge_index[1]
    padlen = _NW * _EPW - _E
    srcp = jnp.concatenate([src, jnp.zeros((padlen,), jnp.int32)]
                           ).reshape(_NW, _NCH, _CH)
    dstp = jnp.concatenate([dst, jnp.full((padlen,), _N, jnp.int32)]
                           ).reshape(_NW, _NCH, _CH)
    ones = jnp.ones((_CH,), _F32)
    zr = jnp.zeros((_RPT, _D), _F32)
    zc = jnp.zeros((_RPT,), _F32)
    xpad = jnp.pad(x, ((0, _NPAD - _N), (0, 0)))
    batchp = jnp.pad(batch, (0, _NPAD - _N),
                     constant_values=_G - 1).reshape(_NPAD, 1)

    srcp2 = srcp.reshape(_NW, _EPW)
    dstp2 = dstp.reshape(_NW, _EPW)
    f, slot = _firsts(batchp)
    slotf = slot.reshape(_NPAD)
    (mark,) = _sc_filter(srcp2, dstp2, slotf, ones, zc)
    sum1, cnt1 = _sc_restricted(xpad, srcp2, dstp2, mark[0], mark[1],
                                ones, zr, zc)
    h = _tc1(sum1[0], sum1[1],
             cnt1[0].reshape(_NPAD, 1), cnt1[1].reshape(_NPAD, 1),
             xpad, W_l0.T, b_l0.reshape(1, _D), W_r0.T)
    sum2, cnt2 = _sc_agg2(h, srcp2, dstp2, slotf, ones, zr, zc)
    return _tcf(f, sum2[0, :_G], sum2[1, :_G],
                cnt2[0, :_G].reshape(_G, 1), cnt2[1, :_G].reshape(_G, 1),
                h, W_l1.T, b_l1.reshape(1, _D), W_r1.T)


# speedup vs baseline: 11.1617x; 1.0032x over previous
import jax
import jax.numpy as jnp
from jax import lax
from jax.experimental import pallas as pl
from jax.experimental.pallas import tpu as pltpu
from jax.experimental.pallas import tpu_sc as plsc

_N = 10000
_E = 320000
_D = 128
_G = 64
_NPAD = 10240
_NSUB = 16
_NCORE = 2
_NW = _NCORE * _NSUB
_CH = 128
_EPW = 10240
_NCH = _EPW // _CH
_RPT = _NPAD // _NSUB

_F32 = jnp.float32


def _sc_agg_body(feat_hbm, srcp_hbm, dstp_hbm, ones_hbm, zr_hbm, zc_hbm,
                 sum_out, cnt_out,
                 src_all, dst_all, rbuf0, rbuf1, ones_v, gsem0, gsem1,
                 acc, cntacc):
    c = lax.axis_index("c")
    s = lax.axis_index("s")
    wid = c * _NSUB + s
    pltpu.sync_copy(ones_hbm, ones_v)
    base = s * _RPT
    pltpu.sync_copy(zr_hbm, acc.at[pl.ds(base, _RPT)])
    pltpu.sync_copy(zc_hbm, cntacc.at[pl.ds(base, _RPT)])
    plsc.subcore_barrier()

    nhc = _NCH // 2

    def half(hh, carry):
        pltpu.sync_copy(srcp_hbm.at[wid, pl.ds(hh * nhc, nhc)], src_all)
        pltpu.sync_copy(dstp_hbm.at[wid, pl.ds(hh * nhc, nhc)], dst_all)
        pltpu.async_copy(feat_hbm.at[src_all.at[0]], rbuf0, gsem0)
        pltpu.async_copy(feat_hbm.at[src_all.at[1]], rbuf1, gsem1)

        def chunk2(t, carry2):
            j0 = 2 * t
            j1 = j0 + 1
            pltpu.make_async_copy(feat_hbm.at[src_all.at[j0]],
                                  rbuf0, gsem0).wait()
            pltpu.sync_copy(rbuf0, acc.at[dst_all.at[j0]], add=True)
            pltpu.sync_copy(ones_v, cntacc.at[dst_all.at[j0]], add=True)

            @pl.when(t < nhc // 2 - 1)
            def _p0():
                pltpu.async_copy(feat_hbm.at[src_all.at[j0 + 2]], rbuf0, gsem0)

            pltpu.make_async_copy(feat_hbm.at[src_all.at[j1]],
                                  rbuf1, gsem1).wait()
            pltpu.sync_copy(rbuf1, acc.at[dst_all.at[j1]], add=True)
            pltpu.sync_copy(ones_v, cntacc.at[dst_all.at[j1]], add=True)

            @pl.when(t < nhc // 2 - 1)
            def _p1():
                pltpu.async_copy(feat_hbm.at[src_all.at[j1 + 2]], rbuf1, gsem1)

            return carry2

        lax.fori_loop(0, nhc // 2, chunk2, 0)
        return carry

    lax.fori_loop(0, 2, half, 0)
    plsc.subcore_barrier()
    pltpu.sync_copy(acc.at[pl.ds(base, _RPT)], sum_out.at[c, pl.ds(base, _RPT)])
    pltpu.sync_copy(cntacc.at[pl.ds(base, _RPT)], cnt_out.at[c, pl.ds(base, _RPT)])


_SL = 128
_CB = 10496
_MTRASH = 10016
_CHR = 64


def _sc_filter_body(srci_hbm, dsti_hbm, slot_hbm, ones_hbm, zc_hbm,
                    mark_out,
                    srcf, dstf, slottab, sidx, ones_v, markacc):
    c = lax.axis_index("c")
    s = lax.axis_index("s")

    base = s * _RPT

    @pl.when(c == 0)
    def _stage():
        pltpu.sync_copy(srci_hbm.at[2 * s], srcf.at[pl.ds(0, _EPW)])
        pltpu.sync_copy(srci_hbm.at[2 * s + 1], srcf.at[pl.ds(_EPW, _EPW)])
        pltpu.sync_copy(dsti_hbm.at[2 * s], dstf.at[pl.ds(0, _EPW)])
        pltpu.sync_copy(dsti_hbm.at[2 * s + 1], dstf.at[pl.ds(_EPW, _EPW)])
        pltpu.sync_copy(slot_hbm, slottab)
        pltpu.sync_copy(ones_hbm, ones_v)
        pltpu.sync_copy(zc_hbm, markacc.at[pl.ds(base, _RPT)])

    plsc.subcore_barrier()

    @pl.when(c == 0)
    def _run():
        def scan(i, fcnt):
            dv = dstf[pl.ds(i * 16, 16)]
            sl = plsc.load_gather(slottab, [dv])
            m = sl >= 0
            sv = srcf[pl.ds(i * 16, 16)]
            plsc.store_compressed(srcf.at[pl.ds(fcnt, 16)], sv, mask=m)
            return fcnt + jnp.sum(m.astype(jnp.int32))

        fcnt = lax.fori_loop(0, 2 * _EPW // 16, scan, 0)

        def appf(i, fcnt):
            sl = slottab[pl.ds(base + i * 16, 16)]
            m = sl >= 0
            ids = (jnp.full((16,), base + i * 16, jnp.int32)
                   + lax.iota(jnp.int32, 16))
            plsc.store_compressed(srcf.at[pl.ds(fcnt, 16)], ids, mask=m)
            return fcnt + jnp.sum(m.astype(jnp.int32))

        fcnt = lax.fori_loop(0, _RPT // 16, appf, fcnt)

        off = (fcnt // 16) * 16
        for k in range(_CH // 16 + 1):
            lanes = (jnp.full((16,), off + k * 16, jnp.int32)
                     + lax.iota(jnp.int32, 16))
            cur = srcf[pl.ds(off + k * 16, 16)]
            srcf[pl.ds(off + k * 16, 16)] = jnp.where(lanes >= fcnt,
                                                      _MTRASH, cur)

        nch = (fcnt + _CH - 1) // _CH

        def chunk(j, carry):
            for k in range(_CH // 16):
                sidx[pl.ds(k * 16, 16)] = srcf[pl.ds(j * _CH + k * 16, 16)]
            pltpu.sync_copy(ones_v, markacc.at[sidx], add=True)
            return carry

        lax.fori_loop(0, nch, chunk, 0)

    plsc.subcore_barrier()

    @pl.when(c == 0)
    def _out():
        pltpu.sync_copy(markacc.at[pl.ds(base, _RPT)],
                        mark_out.at[pl.ds(base, _RPT)])


_CB2 = 2 * _EPW + 512

_sc_filter = pl.kernel(
    _sc_filter_body,
    out_type=[jax.ShapeDtypeStruct((_NPAD,), _F32)],
    mesh=plsc.VectorSubcoreMesh(core_axis_name="c", subcore_axis_name="s"),
    scratch_types=[
        pltpu.VMEM((_CB2,), jnp.int32),
        pltpu.VMEM((2 * _EPW,), jnp.int32),
        pltpu.VMEM((_NPAD,), jnp.int32),
        pltpu.VMEM((_CH,), jnp.int32),
        pltpu.VMEM((_CH,), _F32),
        pltpu.VMEM_SHARED((_NPAD,), _F32),
    ],
    compiler_params=pltpu.CompilerParams(needs_layout_passes=False),
)


def _sc_restricted_body(feat_hbm, srci_hbm, dsti_hbm, mark_hbm,
                        ones_hbm, zr_hbm, zc_hbm,
                        sum_out, cnt_out,
                        srcf, dstf, needtab, sidx0, sidx1, ones_v,
                        rbuf0, rbuf1, gsem0, gsem1, acc, cntacc):
    c = lax.axis_index("c")
    s = lax.axis_index("s")
    wid = c * _NSUB + s
    pltpu.sync_copy(srci_hbm.at[wid], srcf.at[pl.ds(0, _EPW)])
    pltpu.sync_copy(dsti_hbm.at[wid], dstf.at[pl.ds(0, _EPW)])
    pltpu.sync_copy(mark_hbm, needtab)
    pltpu.sync_copy(ones_hbm, ones_v)
    base = s * _RPT
    pltpu.sync_copy(zr_hbm, acc.at[pl.ds(base, _RPT)])
    pltpu.sync_copy(zc_hbm, cntacc.at[pl.ds(base, _RPT)])
    plsc.subcore_barrier()

    def scan(i, fcnt):
        dv = dstf[pl.ds(i * 16, 16)]
        nv = plsc.load_gather(needtab, [dv])
        m = nv > 0.0
        sv = srcf[pl.ds(i * 16, 16)]
        plsc.store_compressed(srcf.at[pl.ds(fcnt, 16)], sv, mask=m)
        plsc.store_compressed(dstf.at[pl.ds(fcnt, 16)], dv, mask=m)
        return fcnt + jnp.sum(m.astype(jnp.int32))

    fcnt = lax.fori_loop(0, _EPW // 16, scan, 0)

    off = (fcnt // 16) * 16
    for k in range(_CHR // 16 + 1):
        lanes = jnp.full((16,), off + k * 16, jnp.int32) + lax.iota(jnp.int32, 16)
        mpad = lanes >= fcnt
        cs = srcf[pl.ds(off + k * 16, 16)]
        srcf[pl.ds(off + k * 16, 16)] = jnp.where(mpad, 0, cs)
        cd = dstf[pl.ds(off + k * 16, 16)]
        dstf[pl.ds(off + k * 16, 16)] = jnp.where(mpad, _N, cd)

    nch = (fcnt + _CHR - 1) // _CHR

    @pl.when(nch > 0)
    def _g0():
        pltpu.async_copy(feat_hbm.at[srcf.at[pl.ds(0, _CHR)]], rbuf0, gsem0)

    @pl.when(nch > 1)
    def _g1():
        pltpu.async_copy(feat_hbm.at[srcf.at[pl.ds(_CHR, _CHR)]], rbuf1, gsem1)

    def chunk2(t, carry):
        j0 = 2 * t
        j1 = j0 + 1
        pltpu.make_async_copy(feat_hbm.at[srcf.at[pl.ds(j0 * _CHR, _CHR)]],
                              rbuf0, gsem0).wait()
        for k in range(_CHR // 16):
            sidx0[pl.ds(k * 16, 16)] = dstf[pl.ds(j0 * _CHR + k * 16, 16)]
        pltpu.sync_copy(rbuf0, acc.at[sidx0], add=True)
        pltpu.sync_copy(ones_v.at[pl.ds(0, _CHR)], cntacc.at[sidx0], add=True)

        @pl.when(j0 + 2 < nch)
        def _p0():
            pltpu.async_copy(feat_hbm.at[srcf.at[pl.ds((j0 + 2) * _CHR, _CHR)]],
                             rbuf0, gsem0)

        @pl.when(j1 < nch)
        def _odd():
            pltpu.make_async_copy(feat_hbm.at[srcf.at[pl.ds(j1 * _CHR, _CHR)]],
                                  rbuf1, gsem1).wait()
            for k in range(_CHR // 16):
                sidx1[pl.ds(k * 16, 16)] = dstf[pl.ds(j1 * _CHR + k * 16, 16)]
            pltpu.sync_copy(rbuf1, acc.at[sidx1], add=True)
            pltpu.sync_copy(ones_v.at[pl.ds(0, _CHR)], cntacc.at[sidx1],
                            add=True)

            @pl.when(j1 + 2 < nch)
            def _p1():
                pltpu.async_copy(
                    feat_hbm.at[srcf.at[pl.ds((j1 + 2) * _CHR, _CHR)]],
                    rbuf1, gsem1)

        return carry

    lax.fori_loop(0, (nch + 1) // 2, chunk2, 0)
    plsc.subcore_barrier()
    pltpu.sync_copy(acc.at[pl.ds(base, _RPT)], sum_out.at[c, pl.ds(base, _RPT)])
    pltpu.sync_copy(cntacc.at[pl.ds(base, _RPT)],
                    cnt_out.at[c, pl.ds(base, _RPT)])


_sc_restricted = pl.kernel(
    _sc_restricted_body,
    out_type=[jax.ShapeDtypeStruct((_NCORE, _NPAD, _D), _F32),
              jax.ShapeDtypeStruct((_NCORE, _NPAD), _F32)],
    mesh=plsc.VectorSubcoreMesh(core_axis_name="c", subcore_axis_name="s"),
    scratch_types=[
        pltpu.VMEM((_CB,), jnp.int32),
        pltpu.VMEM((_CB,), jnp.int32),
        pltpu.VMEM((_NPAD,), _F32),
        pltpu.VMEM((_CHR,), jnp.int32),
        pltpu.VMEM((_CHR,), jnp.int32),
        pltpu.VMEM((_CH,), _F32),
        pltpu.VMEM((_CHR, _D), _F32),
        pltpu.VMEM((_CHR, _D), _F32),
        pltpu.SemaphoreType.DMA,
        pltpu.SemaphoreType.DMA,
        pltpu.VMEM_SHARED((_NPAD, _D), _F32),
        pltpu.VMEM_SHARED((_NPAD,), _F32),
    ],
    compiler_params=pltpu.CompilerParams(needs_layout_passes=False),
)


def _sc_agg2_body(feat_hbm, srci_hbm, dsti_hbm, slot_hbm, ones_hbm,
                  zr_hbm, zc_hbm,
                  sum_out, cnt_out,
                  srcf, dstf, slottab, sidx, rbuf, ones_v,
                  gsem, acc, cntacc):
    c = lax.axis_index("c")
    s = lax.axis_index("s")
    wid = c * _NSUB + s
    pltpu.sync_copy(srci_hbm.at[wid], srcf.at[pl.ds(0, _EPW)])
    pltpu.sync_copy(dsti_hbm.at[wid], dstf.at[pl.ds(0, _EPW)])
    pltpu.sync_copy(slot_hbm, slottab)
    pltpu.sync_copy(ones_hbm, ones_v)

    @pl.when(s == 0)
    def _zero():
        pltpu.sync_copy(zr_hbm.at[pl.ds(0, _SL)], acc)
        pltpu.sync_copy(zc_hbm.at[pl.ds(0, _SL)], cntacc)

    plsc.subcore_barrier()

    def scan(i, fcnt):
        dv = dstf[pl.ds(i * 16, 16)]
        sl = plsc.load_gather(slottab, [dv])
        m = sl >= 0
        sv = srcf[pl.ds(i * 16, 16)]
        plsc.store_compressed(srcf.at[pl.ds(fcnt, 16)], sv, mask=m)
        plsc.store_compressed(dstf.at[pl.ds(fcnt, 16)], sl, mask=m)
        return fcnt + jnp.sum(m.astype(jnp.int32))

    fcnt = lax.fori_loop(0, _EPW // 16, scan, 0)

    off = (fcnt // 16) * 16
    for k in range(_CH // 16 + 1):
        lanes = (jnp.full((16,), off + k * 16, jnp.int32)
                 + lax.iota(jnp.int32, 16))
        mpad = lanes >= fcnt
        cs = srcf[pl.ds(off + k * 16, 16)]
        srcf[pl.ds(off + k * 16, 16)] = jnp.where(mpad, 0, cs)
        cd = dstf[pl.ds(off + k * 16, 16)]
        dstf[pl.ds(off + k * 16, 16)] = jnp.where(mpad, _G, cd)

    nch = (fcnt + _CH - 1) // _CH

    def chunk(j, carry):
        for k in range(_CH // 16):
            sidx[pl.ds(k * 16, 16)] = dstf[pl.ds(j * _CH + k * 16, 16)]
        pltpu.async_copy(feat_hbm.at[srcf.at[pl.ds(j * _CH, _CH)]],
                         rbuf, gsem).wait()
        pltpu.sync_copy(rbuf, acc.at[sidx], add=True)
        pltpu.sync_copy(ones_v, cntacc.at[sidx], add=True)
        return carry

    lax.fori_loop(0, nch, chunk, 0)
    plsc.subcore_barrier()

    @pl.when(s == 0)
    def _write():
        pltpu.sync_copy(acc, sum_out.at[c])
        pltpu.sync_copy(cntacc, cnt_out.at[c])


_sc_agg2 = pl.kernel(
    _sc_agg2_body,
    out_type=[jax.ShapeDtypeStruct((_NCORE, _SL, _D), _F32),
              jax.ShapeDtypeStruct((_NCORE, _SL), _F32)],
    mesh=plsc.VectorSubcoreMesh(core_axis_name="c", subcore_axis_name="s"),
    scratch_types=[
        pltpu.VMEM((_CB,), jnp.int32),
        pltpu.VMEM((_CB,), jnp.int32),
        pltpu.VMEM((_NPAD,), jnp.int32),
        pltpu.VMEM((_CH,), jnp.int32),
        pltpu.VMEM((_CH, _D), _F32),
        pltpu.VMEM((_CH,), _F32),
        pltpu.SemaphoreType.DMA,
        pltpu.VMEM_SHARED((_SL, _D), _F32),
        pltpu.VMEM_SHARED((_SL,), _F32),
    ],
    compiler_params=pltpu.CompilerParams(needs_layout_passes=False),
)


_sc_agg = pl.kernel(
    _sc_agg_body,
    out_type=[jax.ShapeDtypeStruct((_NCORE, _NPAD, _D), _F32),
              jax.ShapeDtypeStruct((_NCORE, _NPAD), _F32)],
    mesh=plsc.VectorSubcoreMesh(core_axis_name="c", subcore_axis_name="s"),
    scratch_types=[
        pltpu.VMEM((_NCH // 2, _CH), jnp.int32),
        pltpu.VMEM((_NCH // 2, _CH), jnp.int32),
        pltpu.VMEM((_CH, _D), _F32),
        pltpu.VMEM((_CH, _D), _F32),
        pltpu.VMEM((_CH,), _F32),
        pltpu.SemaphoreType.DMA,
        pltpu.SemaphoreType.DMA,
        pltpu.VMEM_SHARED((_NPAD, _D), _F32),
        pltpu.VMEM_SHARED((_NPAD,), _F32),
    ],
    compiler_params=pltpu.CompilerParams(needs_layout_passes=False),
)


_R = 1024

def _dot(a, b):
    return jnp.dot(a, b, preferred_element_type=_F32,
                   precision=lax.Precision.HIGHEST)


def _tc1_body(s0, s1, c0, c1, x, wl, bb, wr, o):
    cnt = jnp.maximum(c0[...] + c1[...], 1.0)
    mean = (s0[...] + s1[...]) / cnt
    h = _dot(mean, wl[...]) + bb[...] + _dot(x[...], wr[...])
    o[...] = jnp.maximum(h, 0.0)


def _tc1(s0, s1, c0, c1, xpad, wlT, b, wrT):
    bs_r = pl.BlockSpec((_R, _D), lambda i: (i, 0))
    bs_c = pl.BlockSpec((_R, 1), lambda i: (i, 0))
    bs_w = pl.BlockSpec((_D, _D), lambda i: (0, 0))
    bs_b = pl.BlockSpec((1, _D), lambda i: (0, 0))
    return pl.pallas_call(
        _tc1_body,
        grid=(_NPAD // _R,),
        in_specs=[bs_r, bs_r, bs_c, bs_c, bs_r, bs_w, bs_b, bs_w],
        out_specs=bs_r,
        out_shape=jax.ShapeDtypeStruct((_NPAD, _D), _F32),
    )(s0, s1, c0, c1, xpad, wlT, b, wrT)


def _firsts_body(b_ref, f_ref, slot_ref):
    g = lax.broadcasted_iota(jnp.int32, (1, _G), 1)
    lt = (b_ref[...] < g).astype(jnp.int32)
    f = jnp.minimum(jnp.sum(lt, axis=0, keepdims=True), _N - 1)
    f_ref[...] = f
    rows = lax.broadcasted_iota(jnp.int32, (_NPAD, 1), 0)
    cand = jnp.where(rows == f, g, _G)
    sl = jnp.min(cand, axis=1, keepdims=True)
    slot_ref[...] = jnp.where(sl == _G, -1, sl)


def _firsts(batchp):
    return pl.pallas_call(
        _firsts_body,
        grid=(1,),
        in_specs=[pl.BlockSpec((_NPAD, 1), lambda i: (0, 0))],
        out_specs=[pl.BlockSpec((1, _G), lambda i: (0, 0)),
                   pl.BlockSpec((_NPAD, 1), lambda i: (0, 0))],
        out_shape=[jax.ShapeDtypeStruct((1, _G), jnp.int32),
                   jax.ShapeDtypeStruct((_NPAD, 1), jnp.int32)],
    )(batchp)


def _tcf_body(f, s0, s1, c0, c1, h, wl, bb, wr, o, acch):
    i = pl.program_id(0)

    @pl.when(i == 0)
    def _init():
        acch[...] = jnp.zeros_like(acch)

    rows = lax.broadcasted_iota(jnp.int32, (_R, 1), 0) + i * _R
    oh = (rows == f[...]).astype(_F32)
    acch[...] += lax.dot_general(oh, h[...], (((0,), (0,)), ((), ())),
                                 preferred_element_type=_F32,
                                 precision=lax.Precision.HIGHEST)

    @pl.when(i == _NPAD // _R - 1)
    def _fin():
        fv = f[...]
        gp = lax.broadcasted_iota(jnp.int32, (_G, _G), 1)
        eqm = jnp.reshape(fv, (_G, 1)) == fv
        dmin = jnp.min(jnp.where(eqm, gp, _G), axis=1, keepdims=True)
        P = (gp == dmin).astype(_F32)
        cnt = jnp.maximum(c0[...] + c1[...], 1.0)
        mean = (s0[...] + s1[...]) / cnt
        mean_sel = _dot(P, mean)
        o[...] = _dot(mean_sel, wl[...]) + bb[...] + _dot(acch[...], wr[...])


def _tcf(f, s0, s1, c0, c1, h, wlT, b, wrT):
    bs_r = pl.BlockSpec((_R, _D), lambda i: (i, 0))
    bs_g = pl.BlockSpec((_G, _D), lambda i: (0, 0))
    bs_c = pl.BlockSpec((_G, 1), lambda i: (0, 0))
    bs_w = pl.BlockSpec((_D, _D), lambda i: (0, 0))
    bs_b = pl.BlockSpec((1, _D), lambda i: (0, 0))
    bs_f = pl.BlockSpec((1, _G), lambda i: (0, 0))
    return pl.pallas_call(
        _tcf_body,
        grid=(_NPAD // _R,),
        in_specs=[bs_f, bs_g, bs_g, bs_c, bs_c, bs_r, bs_w, bs_b, bs_w],
        out_specs=pl.BlockSpec((_G, _D), lambda i: (0, 0)),
        out_shape=jax.ShapeDtypeStruct((_G, _D), _F32),
        scratch_shapes=[pltpu.VMEM((_G, _D), _F32)],
    )(f, s0, s1, c0, c1, h, wlT, b, wrT)


def kernel(x, edge_index, batch, W_l0, b_l0, W_r0, W_l1, b_l1, W_r1):
    src = edge_index[0]
    dst = edge_index[1]
    padlen = _NW * _EPW - _E
    srcp = jnp.concatenate([src, jnp.zeros((padlen,), jnp.int32)]
                           ).reshape(_NW, _NCH, _CH)
    dstp = jnp.concatenate([dst, jnp.full((padlen,), _N, jnp.int32)]
                           ).reshape(_NW, _NCH, _CH)
    ones = jnp.ones((_CH,), _F32)
    zr = jnp.zeros((_RPT, _D), _F32)
    zc = jnp.zeros((_RPT,), _F32)
    xpad = jnp.pad(x, ((0, _NPAD - _N), (0, 0)))
    batchp = jnp.pad(batch, (0, _NPAD - _N),
                     constant_values=_G - 1).reshape(_NPAD, 1)

    srcp2 = srcp.reshape(_NW, _EPW)
    dstp2 = dstp.reshape(_NW, _EPW)
    f, slot = _firsts(batchp)
    slotf = slot.reshape(_NPAD)
    (mark,) = _sc_filter(srcp2, dstp2, slotf, ones, zc)
    sum1, cnt1 = _sc_restricted(xpad, srcp2, dstp2, mark, ones, zr, zc)
    h = _tc1(sum1[0], sum1[1],
             cnt1[0].reshape(_NPAD, 1), cnt1[1].reshape(_NPAD, 1),
             xpad, W_l0.T, b_l0.reshape(1, _D), W_r0.T)
    sum2, cnt2 = _sc_agg2(h, srcp2, dstp2, slotf, ones, zr, zc)
    return _tcf(f, sum2[0, :_G], sum2[1, :_G],
                cnt2[0, :_G].reshape(_G, 1), cnt2[1, :_G].reshape(_G, 1),
                h, W_l1.T, b_l1.reshape(1, _D), W_r1.T)


# speedup vs baseline: 11.1825x; 1.0019x over previous
import jax
import jax.numpy as jnp
from jax import lax
from jax.experimental import pallas as pl
from jax.experimental.pallas import tpu as pltpu
from jax.experimental.pallas import tpu_sc as plsc

_N = 10000
_E = 320000
_D = 128
_G = 64
_NPAD = 10240
_NSUB = 16
_NCORE = 2
_NW = _NCORE * _NSUB
_CH = 128
_EPW = 10240
_NCH = _EPW // _CH
_RPT = _NPAD // _NSUB

_F32 = jnp.float32


def _sc_agg_body(feat_hbm, srcp_hbm, dstp_hbm, ones_hbm, zr_hbm, zc_hbm,
                 sum_out, cnt_out,
                 src_all, dst_all, rbuf0, rbuf1, ones_v, gsem0, gsem1,
                 acc, cntacc):
    c = lax.axis_index("c")
    s = lax.axis_index("s")
    wid = c * _NSUB + s
    pltpu.sync_copy(ones_hbm, ones_v)
    base = s * _RPT
    pltpu.sync_copy(zr_hbm, acc.at[pl.ds(base, _RPT)])
    pltpu.sync_copy(zc_hbm, cntacc.at[pl.ds(base, _RPT)])
    plsc.subcore_barrier()

    nhc = _NCH // 2

    def half(hh, carry):
        pltpu.sync_copy(srcp_hbm.at[wid, pl.ds(hh * nhc, nhc)], src_all)
        pltpu.sync_copy(dstp_hbm.at[wid, pl.ds(hh * nhc, nhc)], dst_all)
        pltpu.async_copy(feat_hbm.at[src_all.at[0]], rbuf0, gsem0)
        pltpu.async_copy(feat_hbm.at[src_all.at[1]], rbuf1, gsem1)

        def chunk2(t, carry2):
            j0 = 2 * t
            j1 = j0 + 1
            pltpu.make_async_copy(feat_hbm.at[src_all.at[j0]],
                                  rbuf0, gsem0).wait()
            pltpu.sync_copy(rbuf0, acc.at[dst_all.at[j0]], add=True)
            pltpu.sync_copy(ones_v, cntacc.at[dst_all.at[j0]], add=True)

            @pl.when(t < nhc // 2 - 1)
            def _p0():
                pltpu.async_copy(feat_hbm.at[src_all.at[j0 + 2]], rbuf0, gsem0)

            pltpu.make_async_copy(feat_hbm.at[src_all.at[j1]],
                                  rbuf1, gsem1).wait()
            pltpu.sync_copy(rbuf1, acc.at[dst_all.at[j1]], add=True)
            pltpu.sync_copy(ones_v, cntacc.at[dst_all.at[j1]], add=True)

            @pl.when(t < nhc // 2 - 1)
            def _p1():
                pltpu.async_copy(feat_hbm.at[src_all.at[j1 + 2]], rbuf1, gsem1)

            return carry2

        lax.fori_loop(0, nhc // 2, chunk2, 0)
        return carry

    lax.fori_loop(0, 2, half, 0)
    plsc.subcore_barrier()
    pltpu.sync_copy(acc.at[pl.ds(base, _RPT)], sum_out.at[c, pl.ds(base, _RPT)])
    pltpu.sync_copy(cntacc.at[pl.ds(base, _RPT)], cnt_out.at[c, pl.ds(base, _RPT)])


_SL = 128
_CB = 10496
_MTRASH = 10016
_CHR = 64


def _sc_filter_body(srci_hbm, dsti_hbm, slot_hbm, ones_hbm, zc_hbm,
                    mark_out,
                    srcf, dstf, slottab, sidx, ones_v, markacc):
    c = lax.axis_index("c")
    s = lax.axis_index("s")

    base = s * _RPT

    @pl.when(c == 0)
    def _stage():
        pltpu.sync_copy(srci_hbm.at[2 * s], srcf.at[pl.ds(0, _EPW)])
        pltpu.sync_copy(srci_hbm.at[2 * s + 1], srcf.at[pl.ds(_EPW, _EPW)])
        pltpu.sync_copy(dsti_hbm.at[2 * s], dstf.at[pl.ds(0, _EPW)])
        pltpu.sync_copy(dsti_hbm.at[2 * s + 1], dstf.at[pl.ds(_EPW, _EPW)])
        pltpu.sync_copy(slot_hbm, slottab)
        pltpu.sync_copy(ones_hbm, ones_v)
        pltpu.sync_copy(zc_hbm, markacc.at[pl.ds(base, _RPT)])

    plsc.subcore_barrier()

    @pl.when(c == 0)
    def _run():
        def scan(i, fcnt):
            dv = dstf[pl.ds(i * 16, 16)]
            sl = plsc.load_gather(slottab, [dv])
            m = sl >= 0
            sv = srcf[pl.ds(i * 16, 16)]
            plsc.store_compressed(srcf.at[pl.ds(fcnt, 16)], sv, mask=m)
            return fcnt + jnp.sum(m.astype(jnp.int32))

        fcnt = lax.fori_loop(0, 2 * _EPW // 16, scan, 0)

        def appf(i, fcnt):
            sl = slottab[pl.ds(base + i * 16, 16)]
            m = sl >= 0
            ids = (jnp.full((16,), base + i * 16, jnp.int32)
                   + lax.iota(jnp.int32, 16))
            plsc.store_compressed(srcf.at[pl.ds(fcnt, 16)], ids, mask=m)
            return fcnt + jnp.sum(m.astype(jnp.int32))

        fcnt = lax.fori_loop(0, _RPT // 16, appf, fcnt)

        off = (fcnt // 16) * 16
        for k in range(_CH // 16 + 1):
            lanes = (jnp.full((16,), off + k * 16, jnp.int32)
                     + lax.iota(jnp.int32, 16))
            cur = srcf[pl.ds(off + k * 16, 16)]
            srcf[pl.ds(off + k * 16, 16)] = jnp.where(lanes >= fcnt,
                                                      _MTRASH, cur)

        nch = (fcnt + _CH - 1) // _CH

        def chunk(j, carry):
            for k in range(_CH // 16):
                sidx[pl.ds(k * 16, 16)] = srcf[pl.ds(j * _CH + k * 16, 16)]
            pltpu.sync_copy(ones_v, markacc.at[sidx], add=True)
            return carry

        lax.fori_loop(0, nch, chunk, 0)

    plsc.subcore_barrier()

    @pl.when(c == 0)
    def _out():
        pltpu.sync_copy(markacc.at[pl.ds(base, _RPT)],
                        mark_out.at[pl.ds(base, _RPT)])


_CB2 = 2 * _EPW + 512

_sc_filter = pl.kernel(
    _sc_filter_body,
    out_type=[jax.ShapeDtypeStruct((_NPAD,), _F32)],
    mesh=plsc.VectorSubcoreMesh(core_axis_name="c", subcore_axis_name="s"),
    scratch_types=[
        pltpu.VMEM((_CB2,), jnp.int32),
        pltpu.VMEM((2 * _EPW,), jnp.int32),
        pltpu.VMEM((_NPAD,), jnp.int32),
        pltpu.VMEM((_CH,), jnp.int32),
        pltpu.VMEM((_CH,), _F32),
        pltpu.VMEM_SHARED((_NPAD,), _F32),
    ],
    compiler_params=pltpu.CompilerParams(needs_layout_passes=False),
)


def _sc_restricted_body(feat_hbm, srci_hbm, dsti_hbm, mark_hbm,
                        ones_hbm, zr_hbm, zc_hbm,
                        sum_out, cnt_out,
                        srcf, dstf, needtab, sidx0, sidx1, ones_v,
                        rbuf0, rbuf1, gsem0, gsem1, acc, cntacc):
    c = lax.axis_index("c")
    s = lax.axis_index("s")
    wid = c * _NSUB + s
    pltpu.sync_copy(srci_hbm.at[wid], srcf.at[pl.ds(0, _EPW)])
    pltpu.sync_copy(dsti_hbm.at[wid], dstf.at[pl.ds(0, _EPW)])
    pltpu.sync_copy(mark_hbm, needtab)
    pltpu.sync_copy(ones_hbm, ones_v)
    base = s * _RPT
    pltpu.sync_copy(zr_hbm, acc.at[pl.ds(base, _RPT)])
    pltpu.sync_copy(zc_hbm, cntacc.at[pl.ds(base, _RPT)])
    plsc.subcore_barrier()

    def scan(i, fcnt):
        dv = dstf[pl.ds(i * 16, 16)]
        nv = plsc.load_gather(needtab, [dv])
        m = nv > 0.0
        sv = srcf[pl.ds(i * 16, 16)]
        plsc.store_compressed(srcf.at[pl.ds(fcnt, 16)], sv, mask=m)
        plsc.store_compressed(dstf.at[pl.ds(fcnt, 16)], dv, mask=m)
        return fcnt + jnp.sum(m.astype(jnp.int32))

    fcnt = lax.fori_loop(0, _EPW // 16, scan, 0)

    off = (fcnt // 16) * 16
    for k in range(_CHR // 16 + 1):
        lanes = jnp.full((16,), off + k * 16, jnp.int32) + lax.iota(jnp.int32, 16)
        mpad = lanes >= fcnt
        cs = srcf[pl.ds(off + k * 16, 16)]
        srcf[pl.ds(off + k * 16, 16)] = jnp.where(mpad, 0, cs)
        cd = dstf[pl.ds(off + k * 16, 16)]
        dstf[pl.ds(off + k * 16, 16)] = jnp.where(mpad, _N, cd)

    nch = (fcnt + _CHR - 1) // _CHR

    @pl.when(nch > 0)
    def _g0():
        pltpu.async_copy(feat_hbm.at[srcf.at[pl.ds(0, _CHR)]], rbuf0, gsem0)

    @pl.when(nch > 1)
    def _g1():
        pltpu.async_copy(feat_hbm.at[srcf.at[pl.ds(_CHR, _CHR)]], rbuf1, gsem1)

    def chunk2(t, carry):
        j0 = 2 * t
        j1 = j0 + 1
        pltpu.make_async_copy(feat_hbm.at[srcf.at[pl.ds(j0 * _CHR, _CHR)]],
                              rbuf0, gsem0).wait()
        for k in range(_CHR // 16):
            sidx0[pl.ds(k * 16, 16)] = dstf[pl.ds(j0 * _CHR + k * 16, 16)]
        pltpu.sync_copy(rbuf0, acc.at[sidx0], add=True)
        pltpu.sync_copy(ones_v.at[pl.ds(0, _CHR)], cntacc.at[sidx0], add=True)

        @pl.when(j0 + 2 < nch)
        def _p0():
            pltpu.async_copy(feat_hbm.at[srcf.at[pl.ds((j0 + 2) * _CHR, _CHR)]],
                             rbuf0, gsem0)

        @pl.when(j1 < nch)
        def _odd():
            pltpu.make_async_copy(feat_hbm.at[srcf.at[pl.ds(j1 * _CHR, _CHR)]],
                                  rbuf1, gsem1).wait()
            for k in range(_CHR // 16):
                sidx1[pl.ds(k * 16, 16)] = dstf[pl.ds(j1 * _CHR + k * 16, 16)]
            pltpu.sync_copy(rbuf1, acc.at[sidx1], add=True)
            pltpu.sync_copy(ones_v.at[pl.ds(0, _CHR)], cntacc.at[sidx1],
                            add=True)

            @pl.when(j1 + 2 < nch)
            def _p1():
                pltpu.async_copy(
                    feat_hbm.at[srcf.at[pl.ds((j1 + 2) * _CHR, _CHR)]],
                    rbuf1, gsem1)

        return carry

    lax.fori_loop(0, (nch + 1) // 2, chunk2, 0)
    plsc.subcore_barrier()
    pltpu.sync_copy(acc.at[pl.ds(base, _RPT)], sum_out.at[c, pl.ds(base, _RPT)])
    pltpu.sync_copy(cntacc.at[pl.ds(base, _RPT)],
                    cnt_out.at[c, pl.ds(base, _RPT)])


_sc_restricted = pl.kernel(
    _sc_restricted_body,
    out_type=[jax.ShapeDtypeStruct((_NCORE, _NPAD, _D), _F32),
              jax.ShapeDtypeStruct((_NCORE, _NPAD), _F32)],
    mesh=plsc.VectorSubcoreMesh(core_axis_name="c", subcore_axis_name="s"),
    scratch_types=[
        pltpu.VMEM((_CB,), jnp.int32),
        pltpu.VMEM((_CB,), jnp.int32),
        pltpu.VMEM((_NPAD,), _F32),
        pltpu.VMEM((_CHR,), jnp.int32),
        pltpu.VMEM((_CHR,), jnp.int32),
        pltpu.VMEM((_CH,), _F32),
        pltpu.VMEM((_CHR, _D), _F32),
        pltpu.VMEM((_CHR, _D), _F32),
        pltpu.SemaphoreType.DMA,
        pltpu.SemaphoreType.DMA,
        pltpu.VMEM_SHARED((_NPAD, _D), _F32),
        pltpu.VMEM_SHARED((_NPAD,), _F32),
    ],
    compiler_params=pltpu.CompilerParams(needs_layout_passes=False),
)


def _sc_agg2_body(feat_hbm, srci_hbm, dsti_hbm, slot_hbm, ones_hbm,
                  zr_hbm, zc_hbm,
                  sum_out, cnt_out,
                  srcf, dstf, fslot, slottab, sidx, rbuf, ones_v,
                  gsem, acc, cntacc):
    c = lax.axis_index("c")
    s = lax.axis_index("s")
    wid = c * _NSUB + s
    pltpu.sync_copy(srci_hbm.at[wid], srcf.at[pl.ds(0, _EPW)])
    pltpu.sync_copy(dsti_hbm.at[wid], dstf.at[pl.ds(0, _EPW)])
    pltpu.sync_copy(slot_hbm, slottab)
    pltpu.sync_copy(ones_hbm, ones_v)

    @pl.when(s == 0)
    def _zero():
        pltpu.sync_copy(zr_hbm.at[pl.ds(0, _SL)], acc)
        pltpu.sync_copy(zc_hbm.at[pl.ds(0, _SL)], cntacc)

    plsc.subcore_barrier()

    def scan(i, fcnt):
        dv = dstf[pl.ds(i * 16, 16)]
        sl = plsc.load_gather(slottab, [dv])
        m = sl >= 0
        sv = srcf[pl.ds(i * 16, 16)]
        plsc.store_compressed(srcf.at[pl.ds(fcnt, 16)], sv, mask=m)
        plsc.store_compressed(fslot.at[pl.ds(fcnt, 16)], sl, mask=m)
        return fcnt + jnp.sum(m.astype(jnp.int32))

    fcnt = lax.fori_loop(0, _EPW // 16, scan, 0)

    off = (fcnt // 16) * 16
    for k in range(_CH // 16 + 1):
        lanes = (jnp.full((16,), off + k * 16, jnp.int32)
                 + lax.iota(jnp.int32, 16))
        mpad = lanes >= fcnt
        cs = srcf[pl.ds(off + k * 16, 16)]
        srcf[pl.ds(off + k * 16, 16)] = jnp.where(mpad, 0, cs)
        cd = fslot[pl.ds(off + k * 16, 16)]
        fslot[pl.ds(off + k * 16, 16)] = jnp.where(mpad, _G, cd)

    nch = (fcnt + _CH - 1) // _CH

    def chunk(j, carry):
        for k in range(_CH // 16):
            sidx[pl.ds(k * 16, 16)] = fslot[pl.ds(j * _CH + k * 16, 16)]
        pltpu.async_copy(feat_hbm.at[srcf.at[pl.ds(j * _CH, _CH)]],
                         rbuf, gsem).wait()
        pltpu.sync_copy(rbuf, acc.at[sidx], add=True)
        pltpu.sync_copy(ones_v, cntacc.at[sidx], add=True)
        return carry

    lax.fori_loop(0, nch, chunk, 0)
    plsc.subcore_barrier()

    @pl.when(s == 0)
    def _write():
        pltpu.sync_copy(acc, sum_out.at[c])
        pltpu.sync_copy(cntacc, cnt_out.at[c])


_sc_agg2 = pl.kernel(
    _sc_agg2_body,
    out_type=[jax.ShapeDtypeStruct((_NCORE, _SL, _D), _F32),
              jax.ShapeDtypeStruct((_NCORE, _SL), _F32)],
    mesh=plsc.VectorSubcoreMesh(core_axis_name="c", subcore_axis_name="s"),
    scratch_types=[
        pltpu.VMEM((_CB,), jnp.int32),
        pltpu.VMEM((_EPW,), jnp.int32),
        pltpu.VMEM((_CB,), jnp.int32),
        pltpu.VMEM((_NPAD,), jnp.int32),
        pltpu.VMEM((_CH,), jnp.int32),
        pltpu.VMEM((_CH, _D), _F32),
        pltpu.VMEM((_CH,), _F32),
        pltpu.SemaphoreType.DMA,
        pltpu.VMEM_SHARED((_SL, _D), _F32),
        pltpu.VMEM_SHARED((_SL,), _F32),
    ],
    compiler_params=pltpu.CompilerParams(needs_layout_passes=False),
)


_sc_agg = pl.kernel(
    _sc_agg_body,
    out_type=[jax.ShapeDtypeStruct((_NCORE, _NPAD, _D), _F32),
              jax.ShapeDtypeStruct((_NCORE, _NPAD), _F32)],
    mesh=plsc.VectorSubcoreMesh(core_axis_name="c", subcore_axis_name="s"),
    scratch_types=[
        pltpu.VMEM((_NCH // 2, _CH), jnp.int32),
        pltpu.VMEM((_NCH // 2, _CH), jnp.int32),
        pltpu.VMEM((_CH, _D), _F32),
        pltpu.VMEM((_CH, _D), _F32),
        pltpu.VMEM((_CH,), _F32),
        pltpu.SemaphoreType.DMA,
        pltpu.SemaphoreType.DMA,
        pltpu.VMEM_SHARED((_NPAD, _D), _F32),
        pltpu.VMEM_SHARED((_NPAD,), _F32),
    ],
    compiler_params=pltpu.CompilerParams(needs_layout_passes=False),
)


_R = 1024

def _dot(a, b):
    return jnp.dot(a, b, preferred_element_type=_F32,
                   precision=lax.Precision.HIGHEST)


def _tc1_body(s0, s1, c0, c1, x, wl, bb, wr, o):
    cnt = jnp.maximum(c0[...] + c1[...], 1.0)
    mean = (s0[...] + s1[...]) / cnt
    h = _dot(mean, wl[...]) + bb[...] + _dot(x[...], wr[...])
    o[...] = jnp.maximum(h, 0.0)


def _tc1(s0, s1, c0, c1, xpad, wlT, b, wrT):
    bs_r = pl.BlockSpec((_R, _D), lambda i: (i, 0))
    bs_c = pl.BlockSpec((_R, 1), lambda i: (i, 0))
    bs_w = pl.BlockSpec((_D, _D), lambda i: (0, 0))
    bs_b = pl.BlockSpec((1, _D), lambda i: (0, 0))
    return pl.pallas_call(
        _tc1_body,
        grid=(_NPAD // _R,),
        in_specs=[bs_r, bs_r, bs_c, bs_c, bs_r, bs_w, bs_b, bs_w],
        out_specs=bs_r,
        out_shape=jax.ShapeDtypeStruct((_NPAD, _D), _F32),
    )(s0, s1, c0, c1, xpad, wlT, b, wrT)


def _firsts_body(b_ref, f_ref, slot_ref):
    g = lax.broadcasted_iota(jnp.int32, (1, _G), 1)
    lt = (b_ref[...] < g).astype(jnp.int32)
    f = jnp.minimum(jnp.sum(lt, axis=0, keepdims=True), _N - 1)
    f_ref[...] = f
    rows = lax.broadcasted_iota(jnp.int32, (_NPAD, 1), 0)
    cand = jnp.where(rows == f, g, _G)
    sl = jnp.min(cand, axis=1, keepdims=True)
    slot_ref[...] = jnp.where(sl == _G, -1, sl)


def _firsts(batchp):
    return pl.pallas_call(
        _firsts_body,
        grid=(1,),
        in_specs=[pl.BlockSpec((_NPAD, 1), lambda i: (0, 0))],
        out_specs=[pl.BlockSpec((1, _G), lambda i: (0, 0)),
                   pl.BlockSpec((_NPAD, 1), lambda i: (0, 0))],
        out_shape=[jax.ShapeDtypeStruct((1, _G), jnp.int32),
                   jax.ShapeDtypeStruct((_NPAD, 1), jnp.int32)],
    )(batchp)


def _tcf_body(f, s0, s1, c0, c1, h, wl, bb, wr, o, acch):
    i = pl.program_id(0)

    @pl.when(i == 0)
    def _init():
        acch[...] = jnp.zeros_like(acch)

    rows = lax.broadcasted_iota(jnp.int32, (_R, 1), 0) + i * _R
    oh = (rows == f[...]).astype(_F32)
    acch[...] += lax.dot_general(oh, h[...], (((0,), (0,)), ((), ())),
                                 preferred_element_type=_F32,
                                 precision=lax.Precision.HIGHEST)

    @pl.when(i == _NPAD // _R - 1)
    def _fin():
        fv = f[...]
        gp = lax.broadcasted_iota(jnp.int32, (_G, _G), 1)
        eqm = jnp.reshape(fv, (_G, 1)) == fv
        dmin = jnp.min(jnp.where(eqm, gp, _G), axis=1, keepdims=True)
        P = (gp == dmin).astype(_F32)
        cnt = jnp.maximum(c0[...] + c1[...], 1.0)
        mean = (s0[...] + s1[...]) / cnt
        mean_sel = _dot(P, mean)
        o[...] = _dot(mean_sel, wl[...]) + bb[...] + _dot(acch[...], wr[...])


def _tcf(f, s0, s1, c0, c1, h, wlT, b, wrT):
    bs_r = pl.BlockSpec((_R, _D), lambda i: (i, 0))
    bs_g = pl.BlockSpec((_G, _D), lambda i: (0, 0))
    bs_c = pl.BlockSpec((_G, 1), lambda i: (0, 0))
    bs_w = pl.BlockSpec((_D, _D), lambda i: (0, 0))
    bs_b = pl.BlockSpec((1, _D), lambda i: (0, 0))
    bs_f = pl.BlockSpec((1, _G), lambda i: (0, 0))
    return pl.pallas_call(
        _tcf_body,
        grid=(_NPAD // _R,),
        in_specs=[bs_f, bs_g, bs_g, bs_c, bs_c, bs_r, bs_w, bs_b, bs_w],
        out_specs=pl.BlockSpec((_G, _D), lambda i: (0, 0)),
        out_shape=jax.ShapeDtypeStruct((_G, _D), _F32),
        scratch_shapes=[pltpu.VMEM((_G, _D), _F32)],
    )(f, s0, s1, c0, c1, h, wlT, b, wrT)


def kernel(x, edge_index, batch, W_l0, b_l0, W_r0, W_l1, b_l1, W_r1):
    src = edge_index[0]
    dst = edge_index[1]
    padlen = _NW * _EPW - _E
    srcp = jnp.concatenate([src, jnp.zeros((padlen,), jnp.int32)]
                           ).reshape(_NW, _NCH, _CH)
    dstp = jnp.concatenate([dst, jnp.full((padlen,), _N, jnp.int32)]
                           ).reshape(_NW, _NCH, _CH)
    ones = jnp.ones((_CH,), _F32)
    zr = jnp.zeros((_RPT, _D), _F32)
    zc = jnp.zeros((_RPT,), _F32)
    xpad = jnp.pad(x, ((0, _NPAD - _N), (0, 0)))
    batchp = jnp.pad(batch, (0, _NPAD - _N),
                     constant_values=_G - 1).reshape(_NPAD, 1)

    srcp2 = srcp.reshape(_NW, _EPW)
    dstp2 = dstp.reshape(_NW, _EPW)
    f, slot = _firsts(batchp)
    slotf = slot.reshape(_NPAD)
    (mark,) = _sc_filter(srcp2, dstp2, slotf, ones, zc)
    sum1, cnt1 = _sc_restricted(xpad, srcp2, dstp2, mark, ones, zr, zc)
    h = _tc1(sum1[0], sum1[1],
             cnt1[0].reshape(_NPAD, 1), cnt1[1].reshape(_NPAD, 1),
             xpad, W_l0.T, b_l0.reshape(1, _D), W_r0.T)
    sum2, cnt2 = _sc_agg2(h, srcp2, dstp2, slotf, ones, zr, zc)
    return _tcf(f, sum2[0, :_G], sum2[1, :_G],
                cnt2[0, :_G].reshape(_G, 1), cnt2[1, :_G].reshape(_G, 1),
                h, W_l1.T, b_l1.reshape(1, _D), W_r1.T)


# speedup vs baseline: 11.4173x; 1.0210x over previous
import jax
import jax.numpy as jnp
from jax import lax
from jax.experimental import pallas as pl
from jax.experimental.pallas import tpu as pltpu
from jax.experimental.pallas import tpu_sc as plsc

_N = 10000
_E = 320000
_D = 128
_G = 64
_NPAD = 10240
_NSUB = 16
_NCORE = 2
_NW = _NCORE * _NSUB
_CH = 128
_EPW = 10240
_NCH = _EPW // _CH
_RPT = _NPAD // _NSUB

_F32 = jnp.float32


def _sc_agg_body(feat_hbm, srcp_hbm, dstp_hbm, ones_hbm, zr_hbm, zc_hbm,
                 sum_out, cnt_out,
                 src_all, dst_all, rbuf0, rbuf1, ones_v, gsem0, gsem1,
                 acc, cntacc):
    c = lax.axis_index("c")
    s = lax.axis_index("s")
    wid = c * _NSUB + s
    pltpu.sync_copy(ones_hbm, ones_v)
    base = s * _RPT
    pltpu.sync_copy(zr_hbm, acc.at[pl.ds(base, _RPT)])
    pltpu.sync_copy(zc_hbm, cntacc.at[pl.ds(base, _RPT)])
    plsc.subcore_barrier()

    nhc = _NCH // 2

    def half(hh, carry):
        pltpu.sync_copy(srcp_hbm.at[wid, pl.ds(hh * nhc, nhc)], src_all)
        pltpu.sync_copy(dstp_hbm.at[wid, pl.ds(hh * nhc, nhc)], dst_all)
        pltpu.async_copy(feat_hbm.at[src_all.at[0]], rbuf0, gsem0)
        pltpu.async_copy(feat_hbm.at[src_all.at[1]], rbuf1, gsem1)

        def chunk2(t, carry2):
            j0 = 2 * t
            j1 = j0 + 1
            pltpu.make_async_copy(feat_hbm.at[src_all.at[j0]],
                                  rbuf0, gsem0).wait()
            pltpu.sync_copy(rbuf0, acc.at[dst_all.at[j0]], add=True)
            pltpu.sync_copy(ones_v, cntacc.at[dst_all.at[j0]], add=True)

            @pl.when(t < nhc // 2 - 1)
            def _p0():
                pltpu.async_copy(feat_hbm.at[src_all.at[j0 + 2]], rbuf0, gsem0)

            pltpu.make_async_copy(feat_hbm.at[src_all.at[j1]],
                                  rbuf1, gsem1).wait()
            pltpu.sync_copy(rbuf1, acc.at[dst_all.at[j1]], add=True)
            pltpu.sync_copy(ones_v, cntacc.at[dst_all.at[j1]], add=True)

            @pl.when(t < nhc // 2 - 1)
            def _p1():
                pltpu.async_copy(feat_hbm.at[src_all.at[j1 + 2]], rbuf1, gsem1)

            return carry2

        lax.fori_loop(0, nhc // 2, chunk2, 0)
        return carry

    lax.fori_loop(0, 2, half, 0)
    plsc.subcore_barrier()
    pltpu.sync_copy(acc.at[pl.ds(base, _RPT)], sum_out.at[c, pl.ds(base, _RPT)])
    pltpu.sync_copy(cntacc.at[pl.ds(base, _RPT)], cnt_out.at[c, pl.ds(base, _RPT)])


_SL = 128
_CB = 10496
_MTRASH = 10016
_CHR = 64


def _sc_filter_body(srci_hbm, dsti_hbm, slot_hbm, ones_hbm, zc_hbm,
                    mark_out,
                    srcf, dstf, slottab, sidx, ones_v, markacc):
    c = lax.axis_index("c")
    s = lax.axis_index("s")

    base = s * _RPT

    @pl.when(c == 0)
    def _stage():
        pltpu.sync_copy(srci_hbm.at[2 * s], srcf.at[pl.ds(0, _EPW)])
        pltpu.sync_copy(srci_hbm.at[2 * s + 1], srcf.at[pl.ds(_EPW, _EPW)])
        pltpu.sync_copy(dsti_hbm.at[2 * s], dstf.at[pl.ds(0, _EPW)])
        pltpu.sync_copy(dsti_hbm.at[2 * s + 1], dstf.at[pl.ds(_EPW, _EPW)])
        pltpu.sync_copy(slot_hbm, slottab)
        pltpu.sync_copy(ones_hbm, ones_v)
        pltpu.sync_copy(zc_hbm, markacc.at[pl.ds(base, _RPT)])

    plsc.subcore_barrier()

    @pl.when(c == 0)
    def _run():
        def scan(i, fcnt):
            dv = dstf[pl.ds(i * 16, 16)]
            sl = plsc.load_gather(slottab, [dv])
            m = sl >= 0
            sv = srcf[pl.ds(i * 16, 16)]
            plsc.store_compressed(srcf.at[pl.ds(fcnt, 16)], sv, mask=m)
            return fcnt + jnp.sum(m.astype(jnp.int32))

        fcnt = lax.fori_loop(0, 2 * _EPW // 16, scan, 0)

        def appf(i, fcnt):
            sl = slottab[pl.ds(base + i * 16, 16)]
            m = sl >= 0
            ids = (jnp.full((16,), base + i * 16, jnp.int32)
                   + lax.iota(jnp.int32, 16))
            plsc.store_compressed(srcf.at[pl.ds(fcnt, 16)], ids, mask=m)
            return fcnt + jnp.sum(m.astype(jnp.int32))

        fcnt = lax.fori_loop(0, _RPT // 16, appf, fcnt)

        off = (fcnt // 16) * 16
        for k in range(_CH // 16 + 1):
            lanes = (jnp.full((16,), off + k * 16, jnp.int32)
                     + lax.iota(jnp.int32, 16))
            cur = srcf[pl.ds(off + k * 16, 16)]
            srcf[pl.ds(off + k * 16, 16)] = jnp.where(lanes >= fcnt,
                                                      _MTRASH, cur)

        nch = (fcnt + _CH - 1) // _CH

        def chunk(j, carry):
            for k in range(_CH // 16):
                sidx[pl.ds(k * 16, 16)] = srcf[pl.ds(j * _CH + k * 16, 16)]
            pltpu.sync_copy(ones_v, markacc.at[sidx], add=True)
            return carry

        lax.fori_loop(0, nch, chunk, 0)

    plsc.subcore_barrier()

    @pl.when(c == 0)
    def _out():
        pltpu.sync_copy(markacc.at[pl.ds(base, _RPT)],
                        mark_out.at[pl.ds(base, _RPT)])


_CB2 = 2 * _EPW + 512

_sc_filter = pl.kernel(
    _sc_filter_body,
    out_type=[jax.ShapeDtypeStruct((_NPAD,), _F32)],
    mesh=plsc.VectorSubcoreMesh(core_axis_name="c", subcore_axis_name="s"),
    scratch_types=[
        pltpu.VMEM((_CB2,), jnp.int32),
        pltpu.VMEM((2 * _EPW,), jnp.int32),
        pltpu.VMEM((_NPAD,), jnp.int32),
        pltpu.VMEM((_CH,), jnp.int32),
        pltpu.VMEM((_CH,), _F32),
        pltpu.VMEM_SHARED((_NPAD,), _F32),
    ],
    compiler_params=pltpu.CompilerParams(needs_layout_passes=False),
)


def _sc_restricted_body(feat_hbm, srci_hbm, dsti_hbm, slotf_hbm,
                        ones_hbm, zr_hbm, zc_hbm,
                        sum_out, cnt_out,
                        srcf, dstf, needtab, sidx0, sidx1, ones_v,
                        rbuf0, rbuf1, gsem0, gsem1, acc, cntacc, markacc):
    c = lax.axis_index("c")
    s = lax.axis_index("s")
    wid = c * _NSUB + s
    pltpu.sync_copy(slotf_hbm, needtab)
    pltpu.sync_copy(ones_hbm.at[pl.ds(0, _CHR)], ones_v)
    base = s * _RPT
    pltpu.sync_copy(zr_hbm, acc.at[pl.ds(base, _RPT)])
    pltpu.sync_copy(zc_hbm, cntacc.at[pl.ds(base, _RPT)])
    pltpu.sync_copy(zc_hbm.at[pl.ds(0, _RPT)], markacc.at[pl.ds(base, _RPT)])
    plsc.subcore_barrier()

    for p in range(2):
        slab = 2 * s + p
        pltpu.sync_copy(srci_hbm.at[slab], srcf.at[pl.ds(0, _EPW)])
        pltpu.sync_copy(dsti_hbm.at[slab], dstf.at[pl.ds(0, _EPW)])

        def scana(i, fcnt):
            dv = dstf[pl.ds(i * 16, 16)]
            sl = plsc.load_gather(needtab, [dv])
            m = sl >= 0.0
            sv = srcf[pl.ds(i * 16, 16)]
            plsc.store_compressed(srcf.at[pl.ds(fcnt, 16)], sv, mask=m)
            return fcnt + jnp.sum(m.astype(jnp.int32))

        fcnt = lax.fori_loop(0, _EPW // 16, scana, 0)

        if p == 1:
            def appf(i, fcnt):
                sl = needtab[pl.ds(base + i * 16, 16)]
                m = sl >= 0.0
                ids = (jnp.full((16,), base + i * 16, jnp.int32)
                       + lax.iota(jnp.int32, 16))
                plsc.store_compressed(srcf.at[pl.ds(fcnt, 16)], ids, mask=m)
                return fcnt + jnp.sum(m.astype(jnp.int32))

            fcnt = lax.fori_loop(0, _RPT // 16, appf, fcnt)

        off = (fcnt // 16) * 16
        for k in range(_CHR // 16 + 1):
            lanes = (jnp.full((16,), off + k * 16, jnp.int32)
                     + lax.iota(jnp.int32, 16))
            cur = srcf[pl.ds(off + k * 16, 16)]
            srcf[pl.ds(off + k * 16, 16)] = jnp.where(lanes >= fcnt, 0, cur)

        ncha = (fcnt + _CHR - 1) // _CHR

        def chunka(j, carry):
            for k in range(_CHR // 16):
                sidx0[pl.ds(k * 16, 16)] = srcf[pl.ds(j * _CHR + k * 16, 16)]
            pltpu.sync_copy(ones_v.at[pl.ds(0, _CHR)], markacc.at[sidx0],
                            add=True)
            return carry

        lax.fori_loop(0, ncha, chunka, 0)

    plsc.subcore_barrier()
    pltpu.sync_copy(markacc, needtab)
    pltpu.sync_copy(srci_hbm.at[wid], srcf.at[pl.ds(0, _EPW)])
    pltpu.sync_copy(dsti_hbm.at[wid], dstf.at[pl.ds(0, _EPW)])

    def scan(i, fcnt):
        dv = dstf[pl.ds(i * 16, 16)]
        nv = plsc.load_gather(needtab, [dv])
        m = nv > 0.0
        sv = srcf[pl.ds(i * 16, 16)]
        plsc.store_compressed(srcf.at[pl.ds(fcnt, 16)], sv, mask=m)
        plsc.store_compressed(dstf.at[pl.ds(fcnt, 16)], dv, mask=m)
        return fcnt + jnp.sum(m.astype(jnp.int32))

    fcnt = lax.fori_loop(0, _EPW // 16, scan, 0)

    off = (fcnt // 16) * 16
    for k in range(_CHR // 16 + 1):
        lanes = jnp.full((16,), off + k * 16, jnp.int32) + lax.iota(jnp.int32, 16)
        mpad = lanes >= fcnt
        cs = srcf[pl.ds(off + k * 16, 16)]
        srcf[pl.ds(off + k * 16, 16)] = jnp.where(mpad, 0, cs)
        cd = dstf[pl.ds(off + k * 16, 16)]
        dstf[pl.ds(off + k * 16, 16)] = jnp.where(mpad, _N, cd)

    nch = (fcnt + _CHR - 1) // _CHR

    @pl.when(nch > 0)
    def _g0():
        pltpu.async_copy(feat_hbm.at[srcf.at[pl.ds(0, _CHR)]], rbuf0, gsem0)

    @pl.when(nch > 1)
    def _g1():
        pltpu.async_copy(feat_hbm.at[srcf.at[pl.ds(_CHR, _CHR)]], rbuf1, gsem1)

    def chunk2(t, carry):
        j0 = 2 * t
        j1 = j0 + 1
        pltpu.make_async_copy(feat_hbm.at[srcf.at[pl.ds(j0 * _CHR, _CHR)]],
                              rbuf0, gsem0).wait()
        for k in range(_CHR // 16):
            sidx0[pl.ds(k * 16, 16)] = dstf[pl.ds(j0 * _CHR + k * 16, 16)]
        pltpu.sync_copy(rbuf0, acc.at[sidx0], add=True)
        pltpu.sync_copy(ones_v.at[pl.ds(0, _CHR)], cntacc.at[sidx0], add=True)

        @pl.when(j0 + 2 < nch)
        def _p0():
            pltpu.async_copy(feat_hbm.at[srcf.at[pl.ds((j0 + 2) * _CHR, _CHR)]],
                             rbuf0, gsem0)

        @pl.when(j1 < nch)
        def _odd():
            pltpu.make_async_copy(feat_hbm.at[srcf.at[pl.ds(j1 * _CHR, _CHR)]],
                                  rbuf1, gsem1).wait()
            for k in range(_CHR // 16):
                sidx1[pl.ds(k * 16, 16)] = dstf[pl.ds(j1 * _CHR + k * 16, 16)]
            pltpu.sync_copy(rbuf1, acc.at[sidx1], add=True)
            pltpu.sync_copy(ones_v.at[pl.ds(0, _CHR)], cntacc.at[sidx1],
                            add=True)

            @pl.when(j1 + 2 < nch)
            def _p1():
                pltpu.async_copy(
                    feat_hbm.at[srcf.at[pl.ds((j1 + 2) * _CHR, _CHR)]],
                    rbuf1, gsem1)

        return carry

    lax.fori_loop(0, (nch + 1) // 2, chunk2, 0)
    plsc.subcore_barrier()
    pltpu.sync_copy(acc.at[pl.ds(base, _RPT)], sum_out.at[c, pl.ds(base, _RPT)])
    pltpu.sync_copy(cntacc.at[pl.ds(base, _RPT)],
                    cnt_out.at[c, pl.ds(base, _RPT)])


_sc_restricted = pl.kernel(
    _sc_restricted_body,
    out_type=[jax.ShapeDtypeStruct((_NCORE, _NPAD, _D), _F32),
              jax.ShapeDtypeStruct((_NCORE, _NPAD), _F32)],
    mesh=plsc.VectorSubcoreMesh(core_axis_name="c", subcore_axis_name="s"),
    scratch_types=[
        pltpu.VMEM((_CB,), jnp.int32),
        pltpu.VMEM((10368,), jnp.int32),
        pltpu.VMEM((_NPAD,), _F32),
        pltpu.VMEM((_CHR,), jnp.int32),
        pltpu.VMEM((_CHR,), jnp.int32),
        pltpu.VMEM((_CHR,), _F32),
        pltpu.VMEM((_CHR, _D), _F32),
        pltpu.VMEM((_CHR, _D), _F32),
        pltpu.SemaphoreType.DMA,
        pltpu.SemaphoreType.DMA,
        pltpu.VMEM_SHARED((_NPAD, _D), _F32),
        pltpu.VMEM_SHARED((_NPAD,), _F32),
        pltpu.VMEM_SHARED((_NPAD,), _F32),
    ],
    compiler_params=pltpu.CompilerParams(needs_layout_passes=False),
)


def _sc_agg2_body(feat_hbm, srci_hbm, dsti_hbm, slot_hbm, ones_hbm,
                  zr_hbm, zc_hbm,
                  sum_out, cnt_out,
                  srcf, dstf, fslot, slottab, sidx, rbuf, ones_v,
                  gsem, acc, cntacc):
    c = lax.axis_index("c")
    s = lax.axis_index("s")
    wid = c * _NSUB + s
    pltpu.sync_copy(srci_hbm.at[wid], srcf.at[pl.ds(0, _EPW)])
    pltpu.sync_copy(dsti_hbm.at[wid], dstf.at[pl.ds(0, _EPW)])
    pltpu.sync_copy(slot_hbm, slottab)
    pltpu.sync_copy(ones_hbm, ones_v)

    @pl.when(s == 0)
    def _zero():
        pltpu.sync_copy(zr_hbm.at[pl.ds(0, _SL)], acc)
        pltpu.sync_copy(zc_hbm.at[pl.ds(0, _SL)], cntacc)

    plsc.subcore_barrier()

    def scan(i, fcnt):
        dv = dstf[pl.ds(i * 16, 16)]
        sl = plsc.load_gather(slottab, [dv])
        m = sl >= 0
        sv = srcf[pl.ds(i * 16, 16)]
        plsc.store_compressed(srcf.at[pl.ds(fcnt, 16)], sv, mask=m)
        plsc.store_compressed(fslot.at[pl.ds(fcnt, 16)], sl, mask=m)
        return fcnt + jnp.sum(m.astype(jnp.int32))

    fcnt = lax.fori_loop(0, _EPW // 16, scan, 0)

    off = (fcnt // 16) * 16
    for k in range(_CH // 16 + 1):
        lanes = (jnp.full((16,), off + k * 16, jnp.int32)
                 + lax.iota(jnp.int32, 16))
        mpad = lanes >= fcnt
        cs = srcf[pl.ds(off + k * 16, 16)]
        srcf[pl.ds(off + k * 16, 16)] = jnp.where(mpad, 0, cs)
        cd = fslot[pl.ds(off + k * 16, 16)]
        fslot[pl.ds(off + k * 16, 16)] = jnp.where(mpad, _G, cd)

    nch = (fcnt + _CH - 1) // _CH

    def chunk(j, carry):
        for k in range(_CH // 16):
            sidx[pl.ds(k * 16, 16)] = fslot[pl.ds(j * _CH + k * 16, 16)]
        pltpu.async_copy(feat_hbm.at[srcf.at[pl.ds(j * _CH, _CH)]],
                         rbuf, gsem).wait()
        pltpu.sync_copy(rbuf, acc.at[sidx], add=True)
        pltpu.sync_copy(ones_v, cntacc.at[sidx], add=True)
        return carry

    lax.fori_loop(0, nch, chunk, 0)
    plsc.subcore_barrier()

    @pl.when(s == 0)
    def _write():
        pltpu.sync_copy(acc, sum_out.at[c])
        pltpu.sync_copy(cntacc, cnt_out.at[c])


_sc_agg2 = pl.kernel(
    _sc_agg2_body,
    out_type=[jax.ShapeDtypeStruct((_NCORE, _SL, _D), _F32),
              jax.ShapeDtypeStruct((_NCORE, _SL), _F32)],
    mesh=plsc.VectorSubcoreMesh(core_axis_name="c", subcore_axis_name="s"),
    scratch_types=[
        pltpu.VMEM((_CB,), jnp.int32),
        pltpu.VMEM((_EPW,), jnp.int32),
        pltpu.VMEM((_CB,), jnp.int32),
        pltpu.VMEM((_NPAD,), jnp.int32),
        pltpu.VMEM((_CH,), jnp.int32),
        pltpu.VMEM((_CH, _D), _F32),
        pltpu.VMEM((_CH,), _F32),
        pltpu.SemaphoreType.DMA,
        pltpu.VMEM_SHARED((_SL, _D), _F32),
        pltpu.VMEM_SHARED((_SL,), _F32),
    ],
    compiler_params=pltpu.CompilerParams(needs_layout_passes=False),
)


_sc_agg = pl.kernel(
    _sc_agg_body,
    out_type=[jax.ShapeDtypeStruct((_NCORE, _NPAD, _D), _F32),
              jax.ShapeDtypeStruct((_NCORE, _NPAD), _F32)],
    mesh=plsc.VectorSubcoreMesh(core_axis_name="c", subcore_axis_name="s"),
    scratch_types=[
        pltpu.VMEM((_NCH // 2, _CH), jnp.int32),
        pltpu.VMEM((_NCH // 2, _CH), jnp.int32),
        pltpu.VMEM((_CH, _D), _F32),
        pltpu.VMEM((_CH, _D), _F32),
        pltpu.VMEM((_CH,), _F32),
        pltpu.SemaphoreType.DMA,
        pltpu.SemaphoreType.DMA,
        pltpu.VMEM_SHARED((_NPAD, _D), _F32),
        pltpu.VMEM_SHARED((_NPAD,), _F32),
    ],
    compiler_params=pltpu.CompilerParams(needs_layout_passes=False),
)


_R = 1024

def _dot(a, b):
    return jnp.dot(a, b, preferred_element_type=_F32,
                   precision=lax.Precision.HIGHEST)


def _tc1_body(s0, s1, c0, c1, x, wl, bb, wr, o):
    cnt = jnp.maximum(c0[...] + c1[...], 1.0)
    mean = (s0[...] + s1[...]) / cnt
    h = _dot(mean, wl[...]) + bb[...] + _dot(x[...], wr[...])
    o[...] = jnp.maximum(h, 0.0)


def _tc1(s0, s1, c0, c1, xpad, wlT, b, wrT):
    bs_r = pl.BlockSpec((_R, _D), lambda i: (i, 0))
    bs_c = pl.BlockSpec((_R, 1), lambda i: (i, 0))
    bs_w = pl.BlockSpec((_D, _D), lambda i: (0, 0))
    bs_b = pl.BlockSpec((1, _D), lambda i: (0, 0))
    return pl.pallas_call(
        _tc1_body,
        grid=(_NPAD // _R,),
        in_specs=[bs_r, bs_r, bs_c, bs_c, bs_r, bs_w, bs_b, bs_w],
        out_specs=bs_r,
        out_shape=jax.ShapeDtypeStruct((_NPAD, _D), _F32),
    )(s0, s1, c0, c1, xpad, wlT, b, wrT)


def _firsts_body(b_ref, f_ref, slot_ref, slotf_ref):
    g = lax.broadcasted_iota(jnp.int32, (1, _G), 1)
    lt = (b_ref[...] < g).astype(jnp.int32)
    f = jnp.minimum(jnp.sum(lt, axis=0, keepdims=True), _N - 1)
    f_ref[...] = f
    rows = lax.broadcasted_iota(jnp.int32, (_NPAD, 1), 0)
    cand = jnp.where(rows == f, g, _G)
    sl = jnp.min(cand, axis=1, keepdims=True)
    sl = jnp.where(sl == _G, -1, sl)
    slot_ref[...] = sl
    slotf_ref[...] = sl.astype(_F32)


def _firsts(batchp):
    return pl.pallas_call(
        _firsts_body,
        grid=(1,),
        in_specs=[pl.BlockSpec((_NPAD, 1), lambda i: (0, 0))],
        out_specs=[pl.BlockSpec((1, _G), lambda i: (0, 0)),
                   pl.BlockSpec((_NPAD, 1), lambda i: (0, 0)),
                   pl.BlockSpec((_NPAD, 1), lambda i: (0, 0))],
        out_shape=[jax.ShapeDtypeStruct((1, _G), jnp.int32),
                   jax.ShapeDtypeStruct((_NPAD, 1), jnp.int32),
                   jax.ShapeDtypeStruct((_NPAD, 1), _F32)],
    )(batchp)


def _tcf_body(f, s0, s1, c0, c1, h, wl, bb, wr, o, acch):
    i = pl.program_id(0)

    @pl.when(i == 0)
    def _init():
        acch[...] = jnp.zeros_like(acch)

    rows = lax.broadcasted_iota(jnp.int32, (_R, 1), 0) + i * _R
    oh = (rows == f[...]).astype(_F32)
    acch[...] += lax.dot_general(oh, h[...], (((0,), (0,)), ((), ())),
                                 preferred_element_type=_F32,
                                 precision=lax.Precision.HIGHEST)

    @pl.when(i == _NPAD // _R - 1)
    def _fin():
        fv = f[...]
        gp = lax.broadcasted_iota(jnp.int32, (_G, _G), 1)
        eqm = jnp.reshape(fv, (_G, 1)) == fv
        dmin = jnp.min(jnp.where(eqm, gp, _G), axis=1, keepdims=True)
        P = (gp == dmin).astype(_F32)
        cnt = jnp.maximum(c0[...] + c1[...], 1.0)
        mean = (s0[...] + s1[...]) / cnt
        mean_sel = _dot(P, mean)
        o[...] = _dot(mean_sel, wl[...]) + bb[...] + _dot(acch[...], wr[...])


def _tcf(f, s0, s1, c0, c1, h, wlT, b, wrT):
    bs_r = pl.BlockSpec((_R, _D), lambda i: (i, 0))
    bs_g = pl.BlockSpec((_G, _D), lambda i: (0, 0))
    bs_c = pl.BlockSpec((_G, 1), lambda i: (0, 0))
    bs_w = pl.BlockSpec((_D, _D), lambda i: (0, 0))
    bs_b = pl.BlockSpec((1, _D), lambda i: (0, 0))
    bs_f = pl.BlockSpec((1, _G), lambda i: (0, 0))
    return pl.pallas_call(
        _tcf_body,
        grid=(_NPAD // _R,),
        in_specs=[bs_f, bs_g, bs_g, bs_c, bs_c, bs_r, bs_w, bs_b, bs_w],
        out_specs=pl.BlockSpec((_G, _D), lambda i: (0, 0)),
        out_shape=jax.ShapeDtypeStruct((_G, _D), _F32),
        scratch_shapes=[pltpu.VMEM((_G, _D), _F32)],
    )(f, s0, s1, c0, c1, h, wlT, b, wrT)


def kernel(x, edge_index, batch, W_l0, b_l0, W_r0, W_l1, b_l1, W_r1):
    src = edge_index[0]
    dst = edge_index[1]
    padlen = _NW * _EPW - _E
    srcp = jnp.concatenate([src, jnp.zeros((padlen,), jnp.int32)]
                           ).reshape(_NW, _NCH, _CH)
    dstp = jnp.concatenate([dst, jnp.full((padlen,), _N, jnp.int32)]
                           ).reshape(_NW, _NCH, _CH)
    ones = jnp.ones((_CH,), _F32)
    zr = jnp.zeros((_RPT, _D), _F32)
    zc = jnp.zeros((_RPT,), _F32)
    xpad = jnp.pad(x, ((0, _NPAD - _N), (0, 0)))
    batchp = jnp.pad(batch, (0, _NPAD - _N),
                     constant_values=_G - 1).reshape(_NPAD, 1)

    srcp2 = srcp.reshape(_NW, _EPW)
    dstp2 = dstp.reshape(_NW, _EPW)
    f, slot, slot32 = _firsts(batchp)
    slotf = slot.reshape(_NPAD)
    sum1, cnt1 = _sc_restricted(xpad, srcp2, dstp2, slot32.reshape(_NPAD),
                                ones, zr, zc)
    h = _tc1(sum1[0], sum1[1],
             cnt1[0].reshape(_NPAD, 1), cnt1[1].reshape(_NPAD, 1),
             xpad, W_l0.T, b_l0.reshape(1, _D), W_r0.T)
    sum2, cnt2 = _sc_agg2(h, srcp2, dstp2, slotf, ones, zr, zc)
    return _tcf(f, sum2[0, :_G], sum2[1, :_G],
                cnt2[0, :_G].reshape(_G, 1), cnt2[1, :_G].reshape(_G, 1),
                h, W_l1.T, b_l1.reshape(1, _D), W_r1.T)


# speedup vs baseline: 11.4540x; 1.0032x over previous
import jax
import jax.numpy as jnp
from jax import lax
from jax.experimental import pallas as pl
from jax.experimental.pallas import tpu as pltpu
from jax.experimental.pallas import tpu_sc as plsc

_N = 10000
_E = 320000
_D = 128
_G = 64
_NPAD = 10240
_NSUB = 16
_NCORE = 2
_NW = _NCORE * _NSUB
_CH = 128
_EPW = 10240
_NCH = _EPW // _CH
_RPT = _NPAD // _NSUB

_F32 = jnp.float32


def _sc_agg_body(feat_hbm, srcp_hbm, dstp_hbm, ones_hbm, zr_hbm, zc_hbm,
                 sum_out, cnt_out,
                 src_all, dst_all, rbuf0, rbuf1, ones_v, gsem0, gsem1,
                 acc, cntacc):
    c = lax.axis_index("c")
    s = lax.axis_index("s")
    wid = c * _NSUB + s
    pltpu.sync_copy(ones_hbm, ones_v)
    base = s * _RPT
    pltpu.sync_copy(zr_hbm, acc.at[pl.ds(base, _RPT)])
    pltpu.sync_copy(zc_hbm, cntacc.at[pl.ds(base, _RPT)])
    plsc.subcore_barrier()

    nhc = _NCH // 2

    def half(hh, carry):
        pltpu.sync_copy(srcp_hbm.at[wid, pl.ds(hh * nhc, nhc)], src_all)
        pltpu.sync_copy(dstp_hbm.at[wid, pl.ds(hh * nhc, nhc)], dst_all)
        pltpu.async_copy(feat_hbm.at[src_all.at[0]], rbuf0, gsem0)
        pltpu.async_copy(feat_hbm.at[src_all.at[1]], rbuf1, gsem1)

        def chunk2(t, carry2):
            j0 = 2 * t
            j1 = j0 + 1
            pltpu.make_async_copy(feat_hbm.at[src_all.at[j0]],
                                  rbuf0, gsem0).wait()
            pltpu.sync_copy(rbuf0, acc.at[dst_all.at[j0]], add=True)
            pltpu.sync_copy(ones_v, cntacc.at[dst_all.at[j0]], add=True)

            @pl.when(t < nhc // 2 - 1)
            def _p0():
                pltpu.async_copy(feat_hbm.at[src_all.at[j0 + 2]], rbuf0, gsem0)

            pltpu.make_async_copy(feat_hbm.at[src_all.at[j1]],
                                  rbuf1, gsem1).wait()
            pltpu.sync_copy(rbuf1, acc.at[dst_all.at[j1]], add=True)
            pltpu.sync_copy(ones_v, cntacc.at[dst_all.at[j1]], add=True)

            @pl.when(t < nhc // 2 - 1)
            def _p1():
                pltpu.async_copy(feat_hbm.at[src_all.at[j1 + 2]], rbuf1, gsem1)

            return carry2

        lax.fori_loop(0, nhc // 2, chunk2, 0)
        return carry

    lax.fori_loop(0, 2, half, 0)
    plsc.subcore_barrier()
    pltpu.sync_copy(acc.at[pl.ds(base, _RPT)], sum_out.at[c, pl.ds(base, _RPT)])
    pltpu.sync_copy(cntacc.at[pl.ds(base, _RPT)], cnt_out.at[c, pl.ds(base, _RPT)])


_SL = 128
_CB = 10496
_MTRASH = 10016
_CHR = 64


def _sc_filter_body(srci_hbm, dsti_hbm, slot_hbm, ones_hbm, zc_hbm,
                    mark_out,
                    srcf, dstf, slottab, sidx, ones_v, markacc):
    c = lax.axis_index("c")
    s = lax.axis_index("s")

    base = s * _RPT

    @pl.when(c == 0)
    def _stage():
        pltpu.sync_copy(srci_hbm.at[2 * s], srcf.at[pl.ds(0, _EPW)])
        pltpu.sync_copy(srci_hbm.at[2 * s + 1], srcf.at[pl.ds(_EPW, _EPW)])
        pltpu.sync_copy(dsti_hbm.at[2 * s], dstf.at[pl.ds(0, _EPW)])
        pltpu.sync_copy(dsti_hbm.at[2 * s + 1], dstf.at[pl.ds(_EPW, _EPW)])
        pltpu.sync_copy(slot_hbm, slottab)
        pltpu.sync_copy(ones_hbm, ones_v)
        pltpu.sync_copy(zc_hbm, markacc.at[pl.ds(base, _RPT)])

    plsc.subcore_barrier()

    @pl.when(c == 0)
    def _run():
        def scan(i, fcnt):
            dv = dstf[pl.ds(i * 16, 16)]
            sl = plsc.load_gather(slottab, [dv])
            m = sl >= 0
            sv = srcf[pl.ds(i * 16, 16)]
            plsc.store_compressed(srcf.at[pl.ds(fcnt, 16)], sv, mask=m)
            return fcnt + jnp.sum(m.astype(jnp.int32))

        fcnt = lax.fori_loop(0, 2 * _EPW // 16, scan, 0)

        def appf(i, fcnt):
            sl = slottab[pl.ds(base + i * 16, 16)]
            m = sl >= 0
            ids = (jnp.full((16,), base + i * 16, jnp.int32)
                   + lax.iota(jnp.int32, 16))
            plsc.store_compressed(srcf.at[pl.ds(fcnt, 16)], ids, mask=m)
            return fcnt + jnp.sum(m.astype(jnp.int32))

        fcnt = lax.fori_loop(0, _RPT // 16, appf, fcnt)

        off = (fcnt // 16) * 16
        for k in range(_CH // 16 + 1):
            lanes = (jnp.full((16,), off + k * 16, jnp.int32)
                     + lax.iota(jnp.int32, 16))
            cur = srcf[pl.ds(off + k * 16, 16)]
            srcf[pl.ds(off + k * 16, 16)] = jnp.where(lanes >= fcnt,
                                                      _MTRASH, cur)

        nch = (fcnt + _CH - 1) // _CH

        def chunk(j, carry):
            for k in range(_CH // 16):
                sidx[pl.ds(k * 16, 16)] = srcf[pl.ds(j * _CH + k * 16, 16)]
            pltpu.sync_copy(ones_v, markacc.at[sidx], add=True)
            return carry

        lax.fori_loop(0, nch, chunk, 0)

    plsc.subcore_barrier()

    @pl.when(c == 0)
    def _out():
        pltpu.sync_copy(markacc.at[pl.ds(base, _RPT)],
                        mark_out.at[pl.ds(base, _RPT)])


_CB2 = 2 * _EPW + 512

_sc_filter = pl.kernel(
    _sc_filter_body,
    out_type=[jax.ShapeDtypeStruct((_NPAD,), _F32)],
    mesh=plsc.VectorSubcoreMesh(core_axis_name="c", subcore_axis_name="s"),
    scratch_types=[
        pltpu.VMEM((_CB2,), jnp.int32),
        pltpu.VMEM((2 * _EPW,), jnp.int32),
        pltpu.VMEM((_NPAD,), jnp.int32),
        pltpu.VMEM((_CH,), jnp.int32),
        pltpu.VMEM((_CH,), _F32),
        pltpu.VMEM_SHARED((_NPAD,), _F32),
    ],
    compiler_params=pltpu.CompilerParams(needs_layout_passes=False),
)


def _sc_restricted_body(feat_hbm, srci_hbm, dsti_hbm, slotf_hbm,
                        ones_hbm, zr_hbm, zc_hbm,
                        sum_out, cnt_out,
                        srcf, dstf, needtab, sidx0, sidx1, ones_v,
                        rbuf0, rbuf1, gsem0, gsem1, acc, cntacc, markacc):
    c = lax.axis_index("c")
    s = lax.axis_index("s")
    wid = c * _NSUB + s
    pltpu.sync_copy(slotf_hbm, needtab)
    pltpu.sync_copy(ones_hbm.at[pl.ds(0, _CHR)], ones_v)
    base = s * _RPT
    pltpu.sync_copy(zr_hbm, acc.at[pl.ds(base, _RPT)])
    pltpu.sync_copy(zc_hbm, cntacc.at[pl.ds(base, _RPT)])
    pltpu.sync_copy(zc_hbm.at[pl.ds(0, _RPT)], markacc.at[pl.ds(base, _RPT)])
    plsc.subcore_barrier()

    for p in range(2):
        slab = 2 * s + p
        pltpu.sync_copy(srci_hbm.at[slab], srcf.at[pl.ds(0, _EPW)])
        pltpu.sync_copy(dsti_hbm.at[slab], dstf.at[pl.ds(0, _EPW)])

        def scana(i, fcnt):
            dv = dstf[pl.ds(i * 16, 16)]
            sl = plsc.load_gather(needtab, [dv])
            m = sl >= 0.0
            sv = srcf[pl.ds(i * 16, 16)]
            plsc.store_compressed(srcf.at[pl.ds(fcnt, 16)], sv, mask=m)
            return fcnt + jnp.sum(m.astype(jnp.int32))

        fcnt = lax.fori_loop(0, _EPW // 16, scana, 0)

        if p == 1:
            def appf(i, fcnt):
                sl = needtab[pl.ds(base + i * 16, 16)]
                m = sl >= 0.0
                ids = (jnp.full((16,), base + i * 16, jnp.int32)
                       + lax.iota(jnp.int32, 16))
                plsc.store_compressed(srcf.at[pl.ds(fcnt, 16)], ids, mask=m)
                return fcnt + jnp.sum(m.astype(jnp.int32))

            fcnt = lax.fori_loop(0, _RPT // 16, appf, fcnt)

        off = (fcnt // 16) * 16
        for k in range(_CHR // 16 + 1):
            lanes = (jnp.full((16,), off + k * 16, jnp.int32)
                     + lax.iota(jnp.int32, 16))
            cur = srcf[pl.ds(off + k * 16, 16)]
            srcf[pl.ds(off + k * 16, 16)] = jnp.where(lanes >= fcnt, 0, cur)

        ncha = (fcnt + _CHR - 1) // _CHR

        def chunka(j, carry):
            for k in range(_CHR // 16):
                sidx0[pl.ds(k * 16, 16)] = srcf[pl.ds(j * _CHR + k * 16, 16)]
            pltpu.sync_copy(ones_v.at[pl.ds(0, _CHR)], markacc.at[sidx0],
                            add=True)
            return carry

        lax.fori_loop(0, ncha, chunka, 0)

    plsc.subcore_barrier()
    pltpu.sync_copy(markacc, needtab)
    pltpu.sync_copy(srci_hbm.at[wid], srcf.at[pl.ds(0, _EPW)])
    pltpu.sync_copy(dsti_hbm.at[wid], dstf.at[pl.ds(0, _EPW)])

    def scan(i, fcnt):
        dv = dstf[pl.ds(i * 16, 16)]
        nv = plsc.load_gather(needtab, [dv])
        m = nv > 0.0
        sv = srcf[pl.ds(i * 16, 16)]
        plsc.store_compressed(srcf.at[pl.ds(fcnt, 16)], sv, mask=m)
        plsc.store_compressed(dstf.at[pl.ds(fcnt, 16)], dv, mask=m)
        return fcnt + jnp.sum(m.astype(jnp.int32))

    fcnt = lax.fori_loop(0, _EPW // 16, scan, 0)

    off = (fcnt // 16) * 16
    for k in range(_CHR // 16 + 1):
        lanes = jnp.full((16,), off + k * 16, jnp.int32) + lax.iota(jnp.int32, 16)
        mpad = lanes >= fcnt
        cs = srcf[pl.ds(off + k * 16, 16)]
        srcf[pl.ds(off + k * 16, 16)] = jnp.where(mpad, 0, cs)
        cd = dstf[pl.ds(off + k * 16, 16)]
        dstf[pl.ds(off + k * 16, 16)] = jnp.where(mpad, _N, cd)

    nch = (fcnt + _CHR - 1) // _CHR

    @pl.when(nch > 0)
    def _g0():
        pltpu.async_copy(feat_hbm.at[srcf.at[pl.ds(0, _CHR)]], rbuf0, gsem0)

    @pl.when(nch > 1)
    def _g1():
        pltpu.async_copy(feat_hbm.at[srcf.at[pl.ds(_CHR, _CHR)]], rbuf1, gsem1)

    def chunk2(t, carry):
        j0 = 2 * t
        j1 = j0 + 1
        pltpu.make_async_copy(feat_hbm.at[srcf.at[pl.ds(j0 * _CHR, _CHR)]],
                              rbuf0, gsem0).wait()
        for k in range(_CHR // 16):
            sidx0[pl.ds(k * 16, 16)] = dstf[pl.ds(j0 * _CHR + k * 16, 16)]
        pltpu.sync_copy(rbuf0, acc.at[sidx0], add=True)
        pltpu.sync_copy(ones_v.at[pl.ds(0, _CHR)], cntacc.at[sidx0], add=True)

        @pl.when(j0 + 2 < nch)
        def _p0():
            pltpu.async_copy(feat_hbm.at[srcf.at[pl.ds((j0 + 2) * _CHR, _CHR)]],
                             rbuf0, gsem0)

        @pl.when(j1 < nch)
        def _odd():
            pltpu.make_async_copy(feat_hbm.at[srcf.at[pl.ds(j1 * _CHR, _CHR)]],
                                  rbuf1, gsem1).wait()
            for k in range(_CHR // 16):
                sidx1[pl.ds(k * 16, 16)] = dstf[pl.ds(j1 * _CHR + k * 16, 16)]
            pltpu.sync_copy(rbuf1, acc.at[sidx1], add=True)
            pltpu.sync_copy(ones_v.at[pl.ds(0, _CHR)], cntacc.at[sidx1],
                            add=True)

            @pl.when(j1 + 2 < nch)
            def _p1():
                pltpu.async_copy(
                    feat_hbm.at[srcf.at[pl.ds((j1 + 2) * _CHR, _CHR)]],
                    rbuf1, gsem1)

        return carry

    lax.fori_loop(0, (nch + 1) // 2, chunk2, 0)
    plsc.subcore_barrier()
    pltpu.sync_copy(acc.at[pl.ds(base, _RPT)], sum_out.at[c, pl.ds(base, _RPT)])
    pltpu.sync_copy(cntacc.at[pl.ds(base, _RPT)],
                    cnt_out.at[c, pl.ds(base, _RPT)])


_sc_restricted = pl.kernel(
    _sc_restricted_body,
    out_type=[jax.ShapeDtypeStruct((_NCORE, _NPAD, _D), _F32),
              jax.ShapeDtypeStruct((_NCORE, _NPAD), _F32)],
    mesh=plsc.VectorSubcoreMesh(core_axis_name="c", subcore_axis_name="s"),
    scratch_types=[
        pltpu.VMEM((_CB,), jnp.int32),
        pltpu.VMEM((10368,), jnp.int32),
        pltpu.VMEM((_NPAD,), _F32),
        pltpu.VMEM((_CHR,), jnp.int32),
        pltpu.VMEM((_CHR,), jnp.int32),
        pltpu.VMEM((_CHR,), _F32),
        pltpu.VMEM((_CHR, _D), _F32),
        pltpu.VMEM((_CHR, _D), _F32),
        pltpu.SemaphoreType.DMA,
        pltpu.SemaphoreType.DMA,
        pltpu.VMEM_SHARED((_NPAD, _D), _F32),
        pltpu.VMEM_SHARED((_NPAD,), _F32),
        pltpu.VMEM_SHARED((_NPAD,), _F32),
    ],
    compiler_params=pltpu.CompilerParams(needs_layout_passes=False),
)


def _sc_agg2_body(feat_hbm, srci_hbm, dsti_hbm, slot_hbm,
                  zr_hbm,
                  sum_out,
                  srcf, dstf, fslot, slottab, sidx, rbuf,
                  gsem, acc):
    c = lax.axis_index("c")
    s = lax.axis_index("s")
    wid = c * _NSUB + s
    pltpu.sync_copy(srci_hbm.at[wid], srcf.at[pl.ds(0, _EPW)])
    pltpu.sync_copy(dsti_hbm.at[wid], dstf.at[pl.ds(0, _EPW)])
    pltpu.sync_copy(slot_hbm, slottab)

    @pl.when(s == 0)
    def _zero():
        pltpu.sync_copy(zr_hbm.at[pl.ds(0, _SL)], acc)

    plsc.subcore_barrier()

    def scan(i, fcnt):
        dv = dstf[pl.ds(i * 16, 16)]
        sl = plsc.load_gather(slottab, [dv])
        m = sl >= 0
        sv = srcf[pl.ds(i * 16, 16)]
        plsc.store_compressed(srcf.at[pl.ds(fcnt, 16)], sv, mask=m)
        plsc.store_compressed(fslot.at[pl.ds(fcnt, 16)], sl, mask=m)
        return fcnt + jnp.sum(m.astype(jnp.int32))

    fcnt = lax.fori_loop(0, _EPW // 16, scan, 0)

    off = (fcnt // 16) * 16
    for k in range(_CH // 16 + 1):
        lanes = (jnp.full((16,), off + k * 16, jnp.int32)
                 + lax.iota(jnp.int32, 16))
        mpad = lanes >= fcnt
        cs = srcf[pl.ds(off + k * 16, 16)]
        srcf[pl.ds(off + k * 16, 16)] = jnp.where(mpad, 0, cs)
        cd = fslot[pl.ds(off + k * 16, 16)]
        fslot[pl.ds(off + k * 16, 16)] = jnp.where(mpad, _G, cd)

    nch = (fcnt + _CH - 1) // _CH

    def chunk(j, carry):
        for k in range(_CH // 16):
            sidx[pl.ds(k * 16, 16)] = fslot[pl.ds(j * _CH + k * 16, 16)]
        pltpu.async_copy(feat_hbm.at[srcf.at[pl.ds(j * _CH, _CH)]],
                         rbuf, gsem).wait()
        pltpu.sync_copy(rbuf, acc.at[sidx], add=True)
        return carry

    lax.fori_loop(0, nch, chunk, 0)
    plsc.subcore_barrier()

    @pl.when(s == 0)
    def _write():
        pltpu.sync_copy(acc, sum_out.at[c])


_sc_agg2 = pl.kernel(
    _sc_agg2_body,
    out_type=[jax.ShapeDtypeStruct((_NCORE, _SL, _D), _F32)],
    mesh=plsc.VectorSubcoreMesh(core_axis_name="c", subcore_axis_name="s"),
    scratch_types=[
        pltpu.VMEM((_CB,), jnp.int32),
        pltpu.VMEM((_EPW,), jnp.int32),
        pltpu.VMEM((_CB,), jnp.int32),
        pltpu.VMEM((_NPAD,), jnp.int32),
        pltpu.VMEM((_CH,), jnp.int32),
        pltpu.VMEM((_CH, _D), _F32),
        pltpu.SemaphoreType.DMA,
        pltpu.VMEM_SHARED((_SL, _D), _F32),
    ],
    compiler_params=pltpu.CompilerParams(needs_layout_passes=False),
)


_sc_agg = pl.kernel(
    _sc_agg_body,
    out_type=[jax.ShapeDtypeStruct((_NCORE, _NPAD, _D), _F32),
              jax.ShapeDtypeStruct((_NCORE, _NPAD), _F32)],
    mesh=plsc.VectorSubcoreMesh(core_axis_name="c", subcore_axis_name="s"),
    scratch_types=[
        pltpu.VMEM((_NCH // 2, _CH), jnp.int32),
        pltpu.VMEM((_NCH // 2, _CH), jnp.int32),
        pltpu.VMEM((_CH, _D), _F32),
        pltpu.VMEM((_CH, _D), _F32),
        pltpu.VMEM((_CH,), _F32),
        pltpu.SemaphoreType.DMA,
        pltpu.SemaphoreType.DMA,
        pltpu.VMEM_SHARED((_NPAD, _D), _F32),
        pltpu.VMEM_SHARED((_NPAD,), _F32),
    ],
    compiler_params=pltpu.CompilerParams(needs_layout_passes=False),
)


_R = 1024

def _dot(a, b):
    return jnp.dot(a, b, preferred_element_type=_F32,
                   precision=lax.Precision.HIGHEST)


def _tc1_body(s0, s1, c0, c1, x, wl, bb, wr, o):
    cnt = jnp.maximum(c0[...] + c1[...], 1.0)
    mean = (s0[...] + s1[...]) / cnt
    h = _dot(mean, wl[...]) + bb[...] + _dot(x[...], wr[...])
    o[...] = jnp.maximum(h, 0.0)


def _tc1(s0, s1, c0, c1, xpad, wlT, b, wrT):
    bs_r = pl.BlockSpec((_R, _D), lambda i: (i, 0))
    bs_c = pl.BlockSpec((_R, 1), lambda i: (i, 0))
    bs_w = pl.BlockSpec((_D, _D), lambda i: (0, 0))
    bs_b = pl.BlockSpec((1, _D), lambda i: (0, 0))
    return pl.pallas_call(
        _tc1_body,
        grid=(_NPAD // _R,),
        in_specs=[bs_r, bs_r, bs_c, bs_c, bs_r, bs_w, bs_b, bs_w],
        out_specs=bs_r,
        out_shape=jax.ShapeDtypeStruct((_NPAD, _D), _F32),
    )(s0, s1, c0, c1, xpad, wlT, b, wrT)


def _firsts_body(b_ref, f_ref, slot_ref, slotf_ref):
    g = lax.broadcasted_iota(jnp.int32, (1, _G), 1)
    lt = (b_ref[...] < g).astype(jnp.int32)
    f = jnp.minimum(jnp.sum(lt, axis=0, keepdims=True), _N - 1)
    f_ref[...] = f
    rows = lax.broadcasted_iota(jnp.int32, (_NPAD, 1), 0)
    cand = jnp.where(rows == f, g, _G)
    sl = jnp.min(cand, axis=1, keepdims=True)
    sl = jnp.where(sl == _G, -1, sl)
    slot_ref[...] = sl
    slotf_ref[...] = sl.astype(_F32)


def _firsts(batchp):
    return pl.pallas_call(
        _firsts_body,
        grid=(1,),
        in_specs=[pl.BlockSpec((_NPAD, 1), lambda i: (0, 0))],
        out_specs=[pl.BlockSpec((1, _G), lambda i: (0, 0)),
                   pl.BlockSpec((_NPAD, 1), lambda i: (0, 0)),
                   pl.BlockSpec((_NPAD, 1), lambda i: (0, 0))],
        out_shape=[jax.ShapeDtypeStruct((1, _G), jnp.int32),
                   jax.ShapeDtypeStruct((_NPAD, 1), jnp.int32),
                   jax.ShapeDtypeStruct((_NPAD, 1), _F32)],
    )(batchp)


def _tcf_body(f, s0, s1, c0, c1, h, wl, bb, wr, o, acch, accc):
    i = pl.program_id(0)

    @pl.when(i == 0)
    def _init():
        acch[...] = jnp.zeros_like(acch)
        accc[...] = jnp.zeros_like(accc)

    rows = lax.broadcasted_iota(jnp.int32, (_R, 1), 0) + i * _R
    oh = (rows == f[...]).astype(_F32)

    def gat(b):
        return lax.dot_general(oh, b, (((0,), (0,)), ((), ())),
                               preferred_element_type=_F32,
                               precision=lax.Precision.HIGHEST)

    acch[...] += gat(h[...])
    accc[...] += gat(c0[...] + c1[...])

    @pl.when(i == _NPAD // _R - 1)
    def _fin():
        fv = f[...]
        gp = lax.broadcasted_iota(jnp.int32, (_G, _G), 1)
        eqm = jnp.reshape(fv, (_G, 1)) == fv
        dmin = jnp.min(jnp.where(eqm, gp, _G), axis=1, keepdims=True)
        P = (gp == dmin).astype(_F32)
        num_sel = _dot(P, s0[...] + s1[...])
        mean = num_sel / jnp.maximum(accc[...], 1.0)
        o[...] = _dot(mean, wl[...]) + bb[...] + _dot(acch[...], wr[...])


def _tcf(f, s0, s1, c0, c1, h, wlT, b, wrT):
    bs_r = pl.BlockSpec((_R, _D), lambda i: (i, 0))
    bs_rc = pl.BlockSpec((_R, 1), lambda i: (i, 0))
    bs_g = pl.BlockSpec((_G, _D), lambda i: (0, 0))
    bs_w = pl.BlockSpec((_D, _D), lambda i: (0, 0))
    bs_b = pl.BlockSpec((1, _D), lambda i: (0, 0))
    bs_f = pl.BlockSpec((1, _G), lambda i: (0, 0))
    return pl.pallas_call(
        _tcf_body,
        grid=(_NPAD // _R,),
        in_specs=[bs_f, bs_g, bs_g, bs_rc, bs_rc, bs_r, bs_w, bs_b, bs_w],
        out_specs=pl.BlockSpec((_G, _D), lambda i: (0, 0)),
        out_shape=jax.ShapeDtypeStruct((_G, _D), _F32),
        scratch_shapes=[pltpu.VMEM((_G, _D), _F32),
                        pltpu.VMEM((_G, 1), _F32)],
    )(f, s0, s1, c0, c1, h, wlT, b, wrT)


def kernel(x, edge_index, batch, W_l0, b_l0, W_r0, W_l1, b_l1, W_r1):
    src = edge_index[0]
    dst = edge_index[1]
    padlen = _NW * _EPW - _E
    srcp = jnp.concatenate([src, jnp.zeros((padlen,), jnp.int32)]
                           ).reshape(_NW, _NCH, _CH)
    dstp = jnp.concatenate([dst, jnp.full((padlen,), _N, jnp.int32)]
                           ).reshape(_NW, _NCH, _CH)
    ones = jnp.ones((_CH,), _F32)
    zr = jnp.zeros((_RPT, _D), _F32)
    zc = jnp.zeros((_RPT,), _F32)
    xpad = jnp.pad(x, ((0, _NPAD - _N), (0, 0)))
    batchp = jnp.pad(batch, (0, _NPAD - _N),
                     constant_values=_G - 1).reshape(_NPAD, 1)

    srcp2 = srcp.reshape(_NW, _EPW)
    dstp2 = dstp.reshape(_NW, _EPW)
    f, slot, slot32 = _firsts(batchp)
    slotf = slot.reshape(_NPAD)
    sum1, cnt1 = _sc_restricted(xpad, srcp2, dstp2, slot32.reshape(_NPAD),
                                ones, zr, zc)
    c10 = cnt1[0].reshape(_NPAD, 1)
    c11 = cnt1[1].reshape(_NPAD, 1)
    h = _tc1(sum1[0], sum1[1], c10, c11,
             xpad, W_l0.T, b_l0.reshape(1, _D), W_r0.T)
    (sum2,) = _sc_agg2(h, srcp2, dstp2, slotf, zr)
    return _tcf(f, sum2[0, :_G], sum2[1, :_G], c10, c11,
                h, W_l1.T, b_l1.reshape(1, _D), W_r1.T)
